# Initial kernel scaffold; baseline (speedup 1.0000x reference)
import jax
import jax.numpy as jnp
from jax.experimental import pallas as pl


def kernel(x, edge_index, W1, att_src1, att_dst1, b1, W2, att_src2, att_dst2, b2):
    raise NotImplementedError("write your pallas kernel here")


# speedup vs baseline: 86.6669x; 86.6669x over previous
import functools

import jax
import jax.numpy as jnp
from jax import lax
from jax.experimental import pallas as pl
from jax.experimental.pallas import tpu as pltpu
from jax.experimental.pallas import tpu_sc as plsc

N = 50000
E = 800000
EP = N + E
NP = 50176
N4P = NP * 4
TRASH = N
B = 1024
NSC = 2
NT = 16
ROUNDS = -(-EP // (NSC * NT * B))
EPP = ROUNDS * NSC * NT * B
PER_SC = EPP // NSC
RPT = NP // NT


def _tc_a_body(x2d_ref, w1r_ref, as1_ref, ad1_ref, c1_ref):
    x2d = x2d_ref[...]
    xmax = jnp.max(x2d)
    xmin = jnp.min(x2d)
    w1r = w1r_ref[...]
    s1 = jnp.sum(w1r * as1_ref[...], axis=1)
    d1 = jnp.sum(w1r * ad1_ref[...], axis=1)
    p1 = jnp.maximum(xmax * s1, xmin * s1)
    q1 = jnp.maximum(xmax * d1, xmin * d1)
    m = p1 + q1
    m1 = jnp.where(m > 0, m, 0.2 * m)
    rows = []
    for h in range(4):
        rows.append(jnp.full((1, 128), s1[h], jnp.float32))
    for h in range(4):
        rows.append(jnp.full((1, 128), d1[h], jnp.float32))
    for h in range(4):
        rows.append(jnp.full((1, 128), m1[h], jnp.float32))
    rows.append(jnp.zeros((4, 128), jnp.float32))
    c1_ref[...] = jnp.concatenate(rows, axis=0)


def _tc_a(x2d, w1r, as1r, ad1r):
    return pl.pallas_call(
        _tc_a_body,
        out_shape=jax.ShapeDtypeStruct((16, 128), jnp.float32),
    )(x2d, w1r, as1r, ad1r)


def _sc1_body(src_hbm, dst_hbm, xpad_hbm, c1_hbm, z_hbm, acc_out,
              idxs, idxd, xs, xd, rows, c1v, acc, sem):
    c = lax.axis_index("c")
    s = lax.axis_index("s")
    pltpu.sync_copy(z_hbm.at[pl.ds(s * RPT, RPT), pl.ds(0, 8)],
                    acc.at[pl.ds(s * RPT, RPT)])
    pltpu.sync_copy(c1_hbm, c1v)
    plsc.subcore_barrier()
    s1 = [c1v[h, pl.ds(0, 16)][0] for h in range(4)]
    d1 = [c1v[4 + h, pl.ds(0, 16)][0] for h in range(4)]
    m1 = [c1v[8 + h, pl.ds(0, 16)][0] for h in range(4)]
    iota = lax.broadcasted_iota(jnp.int32, (16,), 0)

    @pl.loop(0, ROUNDS)
    def _round(r):
        base = c * PER_SC + (r * NT + s) * B
        pltpu.sync_copy(src_hbm.at[pl.ds(base, B)], idxs)
        pltpu.sync_copy(dst_hbm.at[pl.ds(base, B)], idxd)
        pltpu.async_copy(xpad_hbm.at[idxs], xs, sem).wait()
        pltpu.async_copy(xpad_hbm.at[idxd], xd, sem).wait()

        @pl.loop(0, B // 16)
        def _grp(g):
            vs = xs[pl.ds(g * 16, 16)]
            vd = xd[pl.ds(g * 16, 16)]
            ridx = g * 16 + iota
            for h in range(4):
                a = vs * s1[h] + vd * d1[h]
                a = jnp.where(a > 0, a, 0.2 * a)
                e = jnp.exp(a - m1[h])
                hv = jnp.full((16,), h, jnp.int32)
                plsc.store_scatter(rows, [ridx, hv], e)
                plsc.store_scatter(rows, [ridx, hv + 4], e * vs)

        pltpu.sync_copy(rows, acc.at[idxd], add=True)

    plsc.subcore_barrier()
    pltpu.sync_copy(acc.at[pl.ds(s * RPT, RPT)],
                    acc_out.at[c, pl.ds(s * RPT, RPT)])


def _sc1(src, dst, xpad, c1, z32):
    mesh = plsc.VectorSubcoreMesh(core_axis_name="c", subcore_axis_name="s")
    f = pl.kernel(
        _sc1_body,
        out_type=jax.ShapeDtypeStruct((NSC, NP, 8), jnp.float32),
        mesh=mesh,
        compiler_params=pltpu.CompilerParams(use_tc_tiling_on_sc=False, needs_layout_passes=False),
        scratch_types=[
            pltpu.VMEM((B,), jnp.int32),
            pltpu.VMEM((B,), jnp.int32),
            pltpu.VMEM((B,), jnp.float32),
            pltpu.VMEM((B,), jnp.float32),
            pltpu.VMEM((B, 8), jnp.float32),
            pltpu.VMEM((16, 128), jnp.float32),
            pltpu.VMEM_SHARED((NP, 8), jnp.float32),
            pltpu.SemaphoreType.DMA,
        ],
    )
    return f(src, dst, xpad, c1, z32)


def _tc_b_body(acc1_ref, w1f_ref, b1_ref, w2_ref, as2f_ref, ad2f_ref,
               h2t_ref, a2c_ref, c2_ref):
    i = pl.program_id(0)
    a = acc1_ref[...]
    den = a[0, :, 0:4] + a[1, :, 0:4] + 1e-16
    ssum = a[0, :, 4:8] + a[1, :, 4:8]
    out1 = ssum / den
    kcol = lax.broadcasted_iota(jnp.int32, (4, 64), 1) // 16
    hrow = lax.broadcasted_iota(jnp.int32, (4, 64), 0)
    p4 = (kcol == hrow).astype(jnp.float32)
    h1 = jnp.dot(out1, p4, preferred_element_type=jnp.float32,
                 precision=lax.Precision.HIGHEST)
    h1 = h1 * w1f_ref[...][None] + b1_ref[...][None]
    h1 = jnp.tanh(h1)
    h2 = jnp.dot(h1, w2_ref[...], preferred_element_type=jnp.float32,
                 precision=lax.Precision.HIGHEST)
    h2t_ref[...] = h2
    kcol2 = lax.broadcasted_iota(jnp.int32, (128, 4), 0) // 32
    hrow2 = lax.broadcasted_iota(jnp.int32, (128, 4), 1)
    q4 = (kcol2 == hrow2).astype(jnp.float32)
    a2s = jnp.dot(h2 * as2f_ref[...][None], q4,
                  preferred_element_type=jnp.float32,
                  precision=lax.Precision.HIGHEST)
    a2d = jnp.dot(h2 * ad2f_ref[...][None], q4,
                  preferred_element_type=jnp.float32,
                  precision=lax.Precision.HIGHEST)
    a2c_ref[...] = jnp.concatenate([a2s, a2d], axis=1)
    pmax = jnp.max(a2s, axis=0)
    qmax = jnp.max(a2d, axis=0)
    rows = [jnp.full((1, 128), pmax[h], jnp.float32) for h in range(4)]
    rows += [jnp.full((1, 128), qmax[h], jnp.float32) for h in range(4)]
    cur = jnp.concatenate(rows, axis=0)

    @pl.when(i == 0)
    def _():
        c2_ref[...] = cur

    @pl.when(i > 0)
    def _():
        c2_ref[...] = jnp.maximum(c2_ref[...], cur)


def _tc_b(acc1, w1f, b1, w2, as2f, ad2f):
    grid = (NP // 1024,)
    return pl.pallas_call(
        _tc_b_body,
        grid=grid,
        in_specs=[
            pl.BlockSpec((NSC, 1024, 8), lambda i: (0, i, 0)),
            pl.BlockSpec((64,), lambda i: (0,)),
            pl.BlockSpec((64,), lambda i: (0,)),
            pl.BlockSpec((64, 128), lambda i: (0, 0)),
            pl.BlockSpec((128,), lambda i: (0,)),
            pl.BlockSpec((128,), lambda i: (0,)),
        ],
        out_specs=[
            pl.BlockSpec((1024, 128), lambda i: (i, 0)),
            pl.BlockSpec((1024, 8), lambda i: (i, 0)),
            pl.BlockSpec((8, 128), lambda i: (0, 0)),
        ],
        out_shape=[
            jax.ShapeDtypeStruct((NP, 128), jnp.float32),
            jax.ShapeDtypeStruct((NP, 8), jnp.float32),
            jax.ShapeDtypeStruct((8, 128), jnp.float32),
        ],
    )(acc1, w1f, b1, w2, as2f, ad2f)


def _sc2_body(src_hbm, dst_hbm, att_hbm, c2_hbm, z_hbm,
              e2t_out, den_out,
              idxs, idxd, asr, adr, rows, e2b, c2v, acc, sem):
    c = lax.axis_index("c")
    s = lax.axis_index("s")
    pltpu.sync_copy(z_hbm.at[pl.ds(s * RPT, RPT), pl.ds(0, 8)],
                    acc.at[pl.ds(s * RPT, RPT)])
    pltpu.sync_copy(z_hbm.at[pl.ds(0, B), pl.ds(0, 8)], rows)
    pltpu.sync_copy(c2_hbm, c2v)
    plsc.subcore_barrier()
    m2 = []
    for h in range(4):
        mm = (c2v[h, pl.ds(0, 16)][0] + c2v[4 + h, pl.ds(0, 16)][0])
        m2.append(jnp.where(mm > 0, mm, 0.2 * mm))
    iota = lax.broadcasted_iota(jnp.int32, (16,), 0)

    @pl.loop(0, ROUNDS)
    def _round(r):
        base = c * PER_SC + (r * NT + s) * B
        pltpu.sync_copy(src_hbm.at[pl.ds(base, B)], idxs)
        pltpu.sync_copy(dst_hbm.at[pl.ds(base, B)], idxd)
        pltpu.async_copy(att_hbm.at[idxs], asr, sem).wait()
        pltpu.async_copy(att_hbm.at[idxd], adr, sem).wait()

        @pl.loop(0, B // 16)
        def _grp(g):
            ridx = g * 16 + iota
            for h in range(4):
                hv = jnp.full((16,), h, jnp.int32)
                av = plsc.load_gather(asr, [ridx, hv])
                bv = plsc.load_gather(adr, [ridx, hv + 4])
                a = av + bv
                a = jnp.where(a > 0, a, 0.2 * a)
                e = jnp.exp(a - m2[h])
                e2b[pl.ds(h * B + g * 16, 16)] = e
                plsc.store_scatter(rows, [ridx, hv], e)

        pltpu.sync_copy(rows, acc.at[idxd], add=True)
        for h in range(4):
            pltpu.sync_copy(e2b.at[pl.ds(h * B, B)],
                            e2t_out.at[h, pl.ds(base, B)])

    plsc.subcore_barrier()
    pltpu.sync_copy(acc.at[pl.ds(s * RPT, RPT)],
                    den_out.at[c, pl.ds(s * RPT, RPT)])


def _sc2(src, dst, att, c2, z32):
    mesh = plsc.VectorSubcoreMesh(core_axis_name="c", subcore_axis_name="s")
    f = pl.kernel(
        _sc2_body,
        out_type=(
            jax.ShapeDtypeStruct((4, EPP), jnp.float32),
            jax.ShapeDtypeStruct((NSC, NP, 8), jnp.float32),
        ),
        mesh=mesh,
        compiler_params=pltpu.CompilerParams(use_tc_tiling_on_sc=False, needs_layout_passes=False),
        scratch_types=[
            pltpu.VMEM((B,), jnp.int32),
            pltpu.VMEM((B,), jnp.int32),
            pltpu.VMEM((B, 8), jnp.float32),
            pltpu.VMEM((B, 8), jnp.float32),
            pltpu.VMEM((B, 8), jnp.float32),
            pltpu.VMEM((4 * B,), jnp.float32),
            pltpu.VMEM((8, 128), jnp.float32),
            pltpu.VMEM_SHARED((NP, 8), jnp.float32),
            pltpu.SemaphoreType.DMA,
        ],
    )
    return f(src, dst, att, c2, z32)


def _sc3_body(src_hbm, dst_hbm, e2t_hbm, h2t_hbm, z_hbm, msum_out,
              idxs, idxd, idx8, ev, rows, acc, sem):
    c = lax.axis_index("c")
    s = lax.axis_index("s")
    for p in range(8):
        pltpu.sync_copy(z_hbm.at[pl.ds(s * RPT, RPT)],
                        acc.at[pl.ds(s * RPT, RPT)])
        plsc.subcore_barrier()

        @pl.loop(0, ROUNDS)
        def _round(r):
            base = c * PER_SC + (r * NT + s) * B
            pltpu.sync_copy(src_hbm.at[pl.ds(base, B)], idxs)
            pltpu.sync_copy(dst_hbm.at[pl.ds(base, B)], idxd)
            pltpu.sync_copy(e2t_hbm.at[p // 2, pl.ds(base, B)], ev)

            @pl.loop(0, B // 16)
            def _gidx(g):
                iv = idxs[pl.ds(g * 16, 16)]
                idx8[pl.ds(g * 16, 16)] = iv * 8 + p

            pltpu.async_copy(h2t_hbm.at[idx8], rows, sem).wait()

            @pl.loop(0, B // 16)
            def _edge(g):
                evec = ev[pl.ds(g * 16, 16)]
                for i in range(16):
                    j = g * 16 + i
                    eb = jnp.full((16,), evec[i], jnp.float32)
                    rows[j, pl.ds(0, 16)] = rows[j, pl.ds(0, 16)] * eb

            pltpu.sync_copy(rows, acc.at[idxd], add=True)

        plsc.subcore_barrier()
        pltpu.sync_copy(acc.at[pl.ds(s * RPT, RPT)],
                        msum_out.at[p, c, pl.ds(s * RPT, RPT)])
        plsc.subcore_barrier()


def _sc3(src, dst, e2t, h2t, z16):
    mesh = plsc.VectorSubcoreMesh(core_axis_name="c", subcore_axis_name="s")
    f = pl.kernel(
        _sc3_body,
        out_type=jax.ShapeDtypeStruct((8, NSC, NP, 16), jnp.float32),
        mesh=mesh,
        compiler_params=pltpu.CompilerParams(use_tc_tiling_on_sc=False, needs_layout_passes=False),
        scratch_types=[
            pltpu.VMEM((B,), jnp.int32),
            pltpu.VMEM((B,), jnp.int32),
            pltpu.VMEM((B,), jnp.int32),
            pltpu.VMEM((B,), jnp.float32),
            pltpu.VMEM((B, 16), jnp.float32),
            pltpu.VMEM_SHARED((NP, 16), jnp.float32),
            pltpu.SemaphoreType.DMA,
        ],
    )
    return f(src, dst, e2t, h2t, z16)


def _tc_c_body(msum_ref, den_ref, b2_ref, out_ref):
    i = pl.program_id(0)
    m = msum_ref[...]
    den = den_ref[...]
    ms = m[:, 0] + m[:, 1]
    dn = den[0, :, 0:4] + den[1, :, 0:4] + 1e-16
    o = jnp.concatenate([ms[p] for p in range(8)], axis=1)
    kcol = lax.broadcasted_iota(jnp.int32, (4, 128), 1) // 32
    hrow = lax.broadcasted_iota(jnp.int32, (4, 128), 0)
    k4 = (kcol == hrow).astype(jnp.float32)
    dnrep = jnp.dot(dn, k4, preferred_element_type=jnp.float32,
                    precision=lax.Precision.HIGHEST)
    o = o / dnrep
    o = jnp.tanh(o + b2_ref[...][None])
    vid = i * 1024 + lax.broadcasted_iota(jnp.int32, (1024, 1), 0)
    o = jnp.where(vid < N, o, 0.0)
    psum = jnp.sum(o, axis=0, keepdims=True)

    @pl.when(i == 0)
    def _():
        out_ref[...] = psum

    @pl.when(i > 0)
    def _():
        out_ref[...] = out_ref[...] + psum

    @pl.when(i == NP // 1024 - 1)
    def _():
        out_ref[...] = out_ref[...] * (1.0 / N)


def _tc_c(msum, den2, b2):
    grid = (NP // 1024,)
    return pl.pallas_call(
        _tc_c_body,
        grid=grid,
        in_specs=[
            pl.BlockSpec((8, NSC, 1024, 16), lambda i: (0, 0, i, 0)),
            pl.BlockSpec((NSC, 1024, 8), lambda i: (0, i, 0)),
            pl.BlockSpec((128,), lambda i: (0,)),
        ],
        out_specs=pl.BlockSpec((1, 128), lambda i: (0, 0)),
        out_shape=jax.ShapeDtypeStruct((1, 128), jnp.float32),
    )(msum, den2, b2)


@jax.jit
def kernel(x, edge_index, W1, att_src1, att_dst1, b1, W2, att_src2,
           att_dst2, b2):
    ei = edge_index.astype(jnp.int32)
    loop = jnp.arange(N, dtype=jnp.int32)
    pad = jnp.full((EPP - EP,), TRASH, jnp.int32)
    src = jnp.concatenate([ei[0], loop, pad])
    dst = jnp.concatenate([ei[1], loop, pad])
    xflat = x[:, 0]
    xpad = jnp.pad(xflat, (0, NP - N), mode="edge")
    x2d = xpad.reshape(392, 128)
    z32 = jnp.zeros((NP, 32), jnp.float32)
    z16 = jnp.zeros((NP, 16), jnp.float32)

    c1 = _tc_a(x2d, W1.reshape(4, 16), att_src1[0], att_dst1[0])
    acc1 = _sc1(src, dst, xpad, c1, z32)
    h2full, a2c, c2 = _tc_b(acc1, W1[0], b1, W2,
                            att_src2.reshape(128), att_dst2.reshape(128))
    h2t = h2full.reshape(NP * 8, 16)
    e2t, den2 = _sc2(src, dst, a2c, c2, z32)
    msum = _sc3(src, dst, e2t, h2t, z16)
    return _tc_c(msum, den2, b2)


# speedup vs baseline: 101.0360x; 1.1658x over previous
import functools

import jax
import jax.numpy as jnp
from jax import lax
from jax.experimental import pallas as pl
from jax.experimental.pallas import tpu as pltpu
from jax.experimental.pallas import tpu_sc as plsc

N = 50000
E = 800000
EP = N + E
NP = 50176
N4P = NP * 4
TRASH = N
B = 1024
NSC = 2
NT = 16
ROUNDS = -(-EP // (NSC * NT * B))
EPP = ROUNDS * NSC * NT * B
PER_SC = EPP // NSC
RPT = NP // NT


def _tc_a_body(x2d_ref, w1r_ref, as1_ref, ad1_ref, c1_ref):
    x2d = x2d_ref[...]
    xmax = jnp.max(x2d)
    xmin = jnp.min(x2d)
    w1r = w1r_ref[...]
    s1 = jnp.sum(w1r * as1_ref[...], axis=1)
    d1 = jnp.sum(w1r * ad1_ref[...], axis=1)
    p1 = jnp.maximum(xmax * s1, xmin * s1)
    q1 = jnp.maximum(xmax * d1, xmin * d1)
    m = p1 + q1
    m1 = jnp.where(m > 0, m, 0.2 * m)
    rows = []
    for h in range(4):
        rows.append(jnp.full((1, 128), s1[h], jnp.float32))
    for h in range(4):
        rows.append(jnp.full((1, 128), d1[h], jnp.float32))
    for h in range(4):
        rows.append(jnp.full((1, 128), m1[h], jnp.float32))
    rows.append(jnp.zeros((4, 128), jnp.float32))
    c1_ref[...] = jnp.concatenate(rows, axis=0)


def _tc_a(x2d, w1r, as1r, ad1r):
    return pl.pallas_call(
        _tc_a_body,
        out_shape=jax.ShapeDtypeStruct((16, 128), jnp.float32),
    )(x2d, w1r, as1r, ad1r)


def _sc1_body(src_hbm, dst_hbm, xpad_hbm, c1_hbm, z_hbm, acc_out,
              idxs, idxd, xs, xd, rows, c1v, acc, sem):
    c = lax.axis_index("c")
    s = lax.axis_index("s")
    pltpu.sync_copy(z_hbm.at[pl.ds(s * RPT, RPT), pl.ds(0, 8)],
                    acc.at[pl.ds(s * RPT, RPT)])
    pltpu.sync_copy(c1_hbm, c1v)
    plsc.subcore_barrier()
    s1 = [c1v[h, pl.ds(0, 16)][0] for h in range(4)]
    d1 = [c1v[4 + h, pl.ds(0, 16)][0] for h in range(4)]
    m1 = [c1v[8 + h, pl.ds(0, 16)][0] for h in range(4)]
    iota = lax.broadcasted_iota(jnp.int32, (16,), 0)

    @pl.loop(0, ROUNDS)
    def _round(r):
        base = c * PER_SC + (r * NT + s) * B
        pltpu.sync_copy(src_hbm.at[pl.ds(base, B)], idxs)
        pltpu.sync_copy(dst_hbm.at[pl.ds(base, B)], idxd)
        pltpu.async_copy(xpad_hbm.at[idxs], xs, sem).wait()
        pltpu.async_copy(xpad_hbm.at[idxd], xd, sem).wait()

        @pl.loop(0, B // 16)
        def _grp(g):
            vs = xs[pl.ds(g * 16, 16)]
            vd = xd[pl.ds(g * 16, 16)]
            ridx = g * 16 + iota
            for h in range(4):
                a = vs * s1[h] + vd * d1[h]
                a = jnp.where(a > 0, a, 0.2 * a)
                e = jnp.exp(a - m1[h])
                hv = jnp.full((16,), h, jnp.int32)
                plsc.store_scatter(rows, [ridx, hv], e)
                plsc.store_scatter(rows, [ridx, hv + 4], e * vs)

        pltpu.sync_copy(rows, acc.at[idxd], add=True)

    plsc.subcore_barrier()
    pltpu.sync_copy(acc.at[pl.ds(s * RPT, RPT)],
                    acc_out.at[c, pl.ds(s * RPT, RPT)])


def _sc1(src, dst, xpad, c1, z32):
    mesh = plsc.VectorSubcoreMesh(core_axis_name="c", subcore_axis_name="s")
    f = pl.kernel(
        _sc1_body,
        out_type=jax.ShapeDtypeStruct((NSC, NP, 8), jnp.float32),
        mesh=mesh,
        compiler_params=pltpu.CompilerParams(use_tc_tiling_on_sc=False, needs_layout_passes=False),
        scratch_types=[
            pltpu.VMEM((B,), jnp.int32),
            pltpu.VMEM((B,), jnp.int32),
            pltpu.VMEM((B,), jnp.float32),
            pltpu.VMEM((B,), jnp.float32),
            pltpu.VMEM((B, 8), jnp.float32),
            pltpu.VMEM((16, 128), jnp.float32),
            pltpu.VMEM_SHARED((NP, 8), jnp.float32),
            pltpu.SemaphoreType.DMA,
        ],
    )
    return f(src, dst, xpad, c1, z32)


def _tc_b_body(acc1_ref, w1f_ref, b1_ref, w2_ref, as2f_ref, ad2f_ref,
               h2t_ref, a2c_ref, c2_ref):
    i = pl.program_id(0)
    a = acc1_ref[...]
    den = a[0, :, 0:4] + a[1, :, 0:4] + 1e-16
    ssum = a[0, :, 4:8] + a[1, :, 4:8]
    out1 = ssum / den
    kcol = lax.broadcasted_iota(jnp.int32, (4, 64), 1) // 16
    hrow = lax.broadcasted_iota(jnp.int32, (4, 64), 0)
    p4 = (kcol == hrow).astype(jnp.float32)
    h1 = jnp.dot(out1, p4, preferred_element_type=jnp.float32,
                 precision=lax.Precision.HIGHEST)
    h1 = h1 * w1f_ref[...][None] + b1_ref[...][None]
    h1 = jnp.tanh(h1)
    h2 = jnp.dot(h1, w2_ref[...], preferred_element_type=jnp.float32,
                 precision=lax.Precision.HIGHEST)
    h2t_ref[...] = h2
    kcol2 = lax.broadcasted_iota(jnp.int32, (128, 4), 0) // 32
    hrow2 = lax.broadcasted_iota(jnp.int32, (128, 4), 1)
    q4 = (kcol2 == hrow2).astype(jnp.float32)
    a2s = jnp.dot(h2 * as2f_ref[...][None], q4,
                  preferred_element_type=jnp.float32,
                  precision=lax.Precision.HIGHEST)
    a2d = jnp.dot(h2 * ad2f_ref[...][None], q4,
                  preferred_element_type=jnp.float32,
                  precision=lax.Precision.HIGHEST)
    a2c_ref[...] = jnp.concatenate([a2s, a2d], axis=1)
    pmax = jnp.max(a2s, axis=0)
    qmax = jnp.max(a2d, axis=0)
    rows = [jnp.full((1, 128), pmax[h], jnp.float32) for h in range(4)]
    rows += [jnp.full((1, 128), qmax[h], jnp.float32) for h in range(4)]
    cur = jnp.concatenate(rows, axis=0)

    @pl.when(i == 0)
    def _():
        c2_ref[...] = cur

    @pl.when(i > 0)
    def _():
        c2_ref[...] = jnp.maximum(c2_ref[...], cur)


def _tc_b(acc1, w1f, b1, w2, as2f, ad2f):
    grid = (NP // 1024,)
    return pl.pallas_call(
        _tc_b_body,
        grid=grid,
        in_specs=[
            pl.BlockSpec((NSC, 1024, 8), lambda i: (0, i, 0)),
            pl.BlockSpec((64,), lambda i: (0,)),
            pl.BlockSpec((64,), lambda i: (0,)),
            pl.BlockSpec((64, 128), lambda i: (0, 0)),
            pl.BlockSpec((128,), lambda i: (0,)),
            pl.BlockSpec((128,), lambda i: (0,)),
        ],
        out_specs=[
            pl.BlockSpec((1024, 128), lambda i: (i, 0)),
            pl.BlockSpec((1024, 8), lambda i: (i, 0)),
            pl.BlockSpec((8, 128), lambda i: (0, 0)),
        ],
        out_shape=[
            jax.ShapeDtypeStruct((NP, 128), jnp.float32),
            jax.ShapeDtypeStruct((NP, 8), jnp.float32),
            jax.ShapeDtypeStruct((8, 128), jnp.float32),
        ],
    )(acc1, w1f, b1, w2, as2f, ad2f)


def _sc2_body(src_hbm, dst_hbm, att_hbm, c2_hbm, z_hbm,
              e2t_out, den_out,
              idxs, idxd, asr, adr, rows, e2b, c2v, acc, sem):
    c = lax.axis_index("c")
    s = lax.axis_index("s")
    pltpu.sync_copy(z_hbm.at[pl.ds(s * RPT, RPT), pl.ds(0, 8)],
                    acc.at[pl.ds(s * RPT, RPT)])
    pltpu.sync_copy(z_hbm.at[pl.ds(0, B), pl.ds(0, 8)], rows)
    pltpu.sync_copy(c2_hbm, c2v)
    plsc.subcore_barrier()
    m2 = []
    for h in range(4):
        mm = (c2v[h, pl.ds(0, 16)][0] + c2v[4 + h, pl.ds(0, 16)][0])
        m2.append(jnp.where(mm > 0, mm, 0.2 * mm))
    iota = lax.broadcasted_iota(jnp.int32, (16,), 0)

    @pl.loop(0, ROUNDS)
    def _round(r):
        base = c * PER_SC + (r * NT + s) * B
        pltpu.sync_copy(src_hbm.at[pl.ds(base, B)], idxs)
        pltpu.sync_copy(dst_hbm.at[pl.ds(base, B)], idxd)
        pltpu.async_copy(att_hbm.at[idxs], asr, sem).wait()
        pltpu.async_copy(att_hbm.at[idxd], adr, sem).wait()

        @pl.loop(0, B // 16)
        def _grp(g):
            ridx = g * 16 + iota
            for h in range(4):
                hv = jnp.full((16,), h, jnp.int32)
                av = plsc.load_gather(asr, [ridx, hv])
                bv = plsc.load_gather(adr, [ridx, hv + 4])
                a = av + bv
                a = jnp.where(a > 0, a, 0.2 * a)
                e = jnp.exp(a - m2[h])
                e2b[pl.ds(h * B + g * 16, 16)] = e
                plsc.store_scatter(rows, [ridx, hv], e)

        pltpu.sync_copy(rows, acc.at[idxd], add=True)
        for h in range(4):
            pltpu.sync_copy(e2b.at[pl.ds(h * B, B)],
                            e2t_out.at[h, pl.ds(base, B)])

    plsc.subcore_barrier()
    pltpu.sync_copy(acc.at[pl.ds(s * RPT, RPT)],
                    den_out.at[c, pl.ds(s * RPT, RPT)])


def _sc2(src, dst, att, c2, z32):
    mesh = plsc.VectorSubcoreMesh(core_axis_name="c", subcore_axis_name="s")
    f = pl.kernel(
        _sc2_body,
        out_type=(
            jax.ShapeDtypeStruct((4, EPP), jnp.float32),
            jax.ShapeDtypeStruct((NSC, NP, 8), jnp.float32),
        ),
        mesh=mesh,
        compiler_params=pltpu.CompilerParams(use_tc_tiling_on_sc=False, needs_layout_passes=False),
        scratch_types=[
            pltpu.VMEM((B,), jnp.int32),
            pltpu.VMEM((B,), jnp.int32),
            pltpu.VMEM((B, 8), jnp.float32),
            pltpu.VMEM((B, 8), jnp.float32),
            pltpu.VMEM((B, 8), jnp.float32),
            pltpu.VMEM((4 * B,), jnp.float32),
            pltpu.VMEM((8, 128), jnp.float32),
            pltpu.VMEM_SHARED((NP, 8), jnp.float32),
            pltpu.SemaphoreType.DMA,
        ],
    )
    return f(src, dst, att, c2, z32)


def _sc3_body(src_hbm, dst_hbm, e2t_hbm, h2t_hbm, z_hbm, msum_out,
              idxs, idxd0, idxd1, idx40, idx41, ev0, ev1, rows0, rows1,
              acc, semg0, semg1, sems0, sems1):
    c = lax.axis_index("c")
    s = lax.axis_index("s")
    idxd = (idxd0, idxd1)
    idx4 = (idx40, idx41)
    ev = (ev0, ev1)
    rows = (rows0, rows1)
    semg = (semg0, semg1)
    sems = (sems0, sems1)

    for p in range(8):
        pltpu.sync_copy(z_hbm.at[pl.ds(s * RPT, RPT)],
                        acc.at[pl.ds(s * RPT, RPT)])
        plsc.subcore_barrier()

        def load_prep(rr, b):
            base = c * PER_SC + (rr * NT + s) * B
            pltpu.sync_copy(src_hbm.at[pl.ds(base, B)], idxs)
            pltpu.sync_copy(dst_hbm.at[pl.ds(base, B)], idxd[b])
            pltpu.sync_copy(e2t_hbm.at[p // 2, pl.ds(base, B)], ev[b])

            @pl.loop(0, B // 16)
            def _gidx(g):
                iv = idxs[pl.ds(g * 16, 16)]
                idx4[b][pl.ds(g * 16, 16)] = iv * 8 + p

            pltpu.async_copy(h2t_hbm.at[idx4[b]], rows[b], semg[b])

        def wait_gather(b):
            pltpu.make_async_copy(h2t_hbm.at[pl.ds(0, B)], rows[b],
                                  semg[b]).wait()

        def multiply(b):
            @pl.loop(0, B // 16)
            def _edge(g):
                evec = ev[b][pl.ds(g * 16, 16)]
                for i in range(16):
                    j = g * 16 + i
                    eb = jnp.full((16,), evec[i], jnp.float32)
                    rows[b][j, pl.ds(0, 16)] = rows[b][j, pl.ds(0, 16)] * eb

        load_prep(0, 0)
        load_prep(1, 1)

        @pl.loop(0, ROUNDS // 2)
        def _round(k):
            for b in range(2):
                rr = 2 * k + b
                wait_gather(b)
                multiply(b)
                pltpu.async_copy(rows[b], acc.at[idxd[b]], sems[b], add=True)

                @pl.when(rr + 2 < ROUNDS)
                def _():
                    pltpu.make_async_copy(rows[b], acc.at[idxd[b]],
                                          sems[b]).wait()
                    load_prep(rr + 2, b)

        pltpu.make_async_copy(rows[0], acc.at[idxd[0]], sems[0]).wait()
        pltpu.make_async_copy(rows[1], acc.at[idxd[1]], sems[1]).wait()
        plsc.subcore_barrier()
        pltpu.sync_copy(acc.at[pl.ds(s * RPT, RPT)],
                        msum_out.at[p, c, pl.ds(s * RPT, RPT)])
        plsc.subcore_barrier()


def _sc3(src, dst, e2t, h2t, z16):
    mesh = plsc.VectorSubcoreMesh(core_axis_name="c", subcore_axis_name="s")
    f = pl.kernel(
        _sc3_body,
        out_type=jax.ShapeDtypeStruct((8, NSC, NP, 16), jnp.float32),
        mesh=mesh,
        compiler_params=pltpu.CompilerParams(use_tc_tiling_on_sc=False, needs_layout_passes=False),
        scratch_types=[
            pltpu.VMEM((B,), jnp.int32),
            pltpu.VMEM((B,), jnp.int32),
            pltpu.VMEM((B,), jnp.int32),
            pltpu.VMEM((B,), jnp.int32),
            pltpu.VMEM((B,), jnp.int32),
            pltpu.VMEM((B,), jnp.float32),
            pltpu.VMEM((B,), jnp.float32),
            pltpu.VMEM((B, 16), jnp.float32),
            pltpu.VMEM((B, 16), jnp.float32),
            pltpu.VMEM_SHARED((NP, 16), jnp.float32),
            pltpu.SemaphoreType.DMA,
            pltpu.SemaphoreType.DMA,
            pltpu.SemaphoreType.DMA,
            pltpu.SemaphoreType.DMA,
        ],
    )
    return f(src, dst, e2t, h2t, z16)


def _tc_c_body(msum_ref, den_ref, b2_ref, out_ref):
    i = pl.program_id(0)
    m = msum_ref[...]
    den = den_ref[...]
    ms = m[:, 0] + m[:, 1]
    dn = den[0, :, 0:4] + den[1, :, 0:4] + 1e-16
    o = jnp.concatenate([ms[p] for p in range(8)], axis=1)
    kcol = lax.broadcasted_iota(jnp.int32, (4, 128), 1) // 32
    hrow = lax.broadcasted_iota(jnp.int32, (4, 128), 0)
    k4 = (kcol == hrow).astype(jnp.float32)
    dnrep = jnp.dot(dn, k4, preferred_element_type=jnp.float32,
                    precision=lax.Precision.HIGHEST)
    o = o / dnrep
    o = jnp.tanh(o + b2_ref[...][None])
    vid = i * 1024 + lax.broadcasted_iota(jnp.int32, (1024, 1), 0)
    o = jnp.where(vid < N, o, 0.0)
    psum = jnp.sum(o, axis=0, keepdims=True)

    @pl.when(i == 0)
    def _():
        out_ref[...] = psum

    @pl.when(i > 0)
    def _():
        out_ref[...] = out_ref[...] + psum

    @pl.when(i == NP // 1024 - 1)
    def _():
        out_ref[...] = out_ref[...] * (1.0 / N)


def _tc_c(msum, den2, b2):
    grid = (NP // 1024,)
    return pl.pallas_call(
        _tc_c_body,
        grid=grid,
        in_specs=[
            pl.BlockSpec((8, NSC, 1024, 16), lambda i: (0, 0, i, 0)),
            pl.BlockSpec((NSC, 1024, 8), lambda i: (0, i, 0)),
            pl.BlockSpec((128,), lambda i: (0,)),
        ],
        out_specs=pl.BlockSpec((1, 128), lambda i: (0, 0)),
        out_shape=jax.ShapeDtypeStruct((1, 128), jnp.float32),
    )(msum, den2, b2)


@jax.jit
def kernel(x, edge_index, W1, att_src1, att_dst1, b1, W2, att_src2,
           att_dst2, b2):
    ei = edge_index.astype(jnp.int32)
    loop = jnp.arange(N, dtype=jnp.int32)
    pad = jnp.full((EPP - EP,), TRASH, jnp.int32)
    src = jnp.concatenate([ei[0], loop, pad])
    dst = jnp.concatenate([ei[1], loop, pad])
    xflat = x[:, 0]
    xpad = jnp.pad(xflat, (0, NP - N), mode="edge")
    x2d = xpad.reshape(392, 128)
    z32 = jnp.zeros((NP, 32), jnp.float32)
    z16 = jnp.zeros((NP, 16), jnp.float32)

    c1 = _tc_a(x2d, W1.reshape(4, 16), att_src1[0], att_dst1[0])
    acc1 = _sc1(src, dst, xpad, c1, z32)
    h2full, a2c, c2 = _tc_b(acc1, W1[0], b1, W2,
                            att_src2.reshape(128), att_dst2.reshape(128))
    h2t = h2full.reshape(NP * 8, 16)
    e2t, den2 = _sc2(src, dst, a2c, c2, z32)
    msum = _sc3(src, dst, e2t, h2t, z16)
    return _tc_c(msum, den2, b2)


# speedup vs baseline: 101.3456x; 1.0031x over previous
import functools

import jax
import jax.numpy as jnp
from jax import lax
from jax.experimental import pallas as pl
from jax.experimental.pallas import tpu as pltpu
from jax.experimental.pallas import tpu_sc as plsc

N = 50000
E = 800000
EP = N + E
NP = 50176
N4P = NP * 4
TRASH = N
B = 1024
NSC = 2
NT = 16
ROUNDS = -(-EP // (NSC * NT * B))
EPP = ROUNDS * NSC * NT * B
PER_SC = EPP // NSC
RPT = NP // NT


def _tc_a_body(x2d_ref, w1r_ref, as1_ref, ad1_ref, c1_ref):
    x2d = x2d_ref[...]
    xmax = jnp.max(x2d)
    xmin = jnp.min(x2d)
    w1r = w1r_ref[...]
    s1 = jnp.sum(w1r * as1_ref[...], axis=1)
    d1 = jnp.sum(w1r * ad1_ref[...], axis=1)
    p1 = jnp.maximum(xmax * s1, xmin * s1)
    q1 = jnp.maximum(xmax * d1, xmin * d1)
    m = p1 + q1
    m1 = jnp.where(m > 0, m, 0.2 * m)
    rows = []
    for h in range(4):
        rows.append(jnp.full((1, 128), s1[h], jnp.float32))
    for h in range(4):
        rows.append(jnp.full((1, 128), d1[h], jnp.float32))
    for h in range(4):
        rows.append(jnp.full((1, 128), m1[h], jnp.float32))
    rows.append(jnp.zeros((4, 128), jnp.float32))
    c1_ref[...] = jnp.concatenate(rows, axis=0)


def _tc_a(x2d, w1r, as1r, ad1r):
    return pl.pallas_call(
        _tc_a_body,
        out_shape=jax.ShapeDtypeStruct((16, 128), jnp.float32),
    )(x2d, w1r, as1r, ad1r)


def _sc1_body(src_hbm, dst_hbm, xpad_hbm, c1_hbm, z_hbm, acc_out,
              idxs, idxd, xs, xd, rows, c1v, acc, sem):
    c = lax.axis_index("c")
    s = lax.axis_index("s")
    pltpu.sync_copy(z_hbm.at[pl.ds(s * RPT, RPT), pl.ds(0, 8)],
                    acc.at[pl.ds(s * RPT, RPT)])
    pltpu.sync_copy(c1_hbm, c1v)
    plsc.subcore_barrier()
    s1 = [c1v[h, pl.ds(0, 16)][0] for h in range(4)]
    d1 = [c1v[4 + h, pl.ds(0, 16)][0] for h in range(4)]
    m1 = [c1v[8 + h, pl.ds(0, 16)][0] for h in range(4)]
    iota = lax.broadcasted_iota(jnp.int32, (16,), 0)

    @pl.loop(0, ROUNDS)
    def _round(r):
        base = c * PER_SC + (r * NT + s) * B
        pltpu.sync_copy(src_hbm.at[pl.ds(base, B)], idxs)
        pltpu.sync_copy(dst_hbm.at[pl.ds(base, B)], idxd)
        pltpu.async_copy(xpad_hbm.at[idxs], xs, sem).wait()
        pltpu.async_copy(xpad_hbm.at[idxd], xd, sem).wait()

        @pl.loop(0, B // 16)
        def _grp(g):
            vs = xs[pl.ds(g * 16, 16)]
            vd = xd[pl.ds(g * 16, 16)]
            ridx = g * 16 + iota
            for h in range(4):
                a = vs * s1[h] + vd * d1[h]
                a = jnp.where(a > 0, a, 0.2 * a)
                e = jnp.exp(a - m1[h])
                hv = jnp.full((16,), h, jnp.int32)
                plsc.store_scatter(rows, [ridx, hv], e)
                plsc.store_scatter(rows, [ridx, hv + 4], e * vs)

        pltpu.sync_copy(rows, acc.at[idxd], add=True)

    plsc.subcore_barrier()
    pltpu.sync_copy(acc.at[pl.ds(s * RPT, RPT)],
                    acc_out.at[c, pl.ds(s * RPT, RPT)])


def _sc1(src, dst, xpad, c1, z32):
    mesh = plsc.VectorSubcoreMesh(core_axis_name="c", subcore_axis_name="s")
    f = pl.kernel(
        _sc1_body,
        out_type=jax.ShapeDtypeStruct((NSC, NP, 8), jnp.float32),
        mesh=mesh,
        compiler_params=pltpu.CompilerParams(use_tc_tiling_on_sc=False, needs_layout_passes=False),
        scratch_types=[
            pltpu.VMEM((B,), jnp.int32),
            pltpu.VMEM((B,), jnp.int32),
            pltpu.VMEM((B,), jnp.float32),
            pltpu.VMEM((B,), jnp.float32),
            pltpu.VMEM((B, 8), jnp.float32),
            pltpu.VMEM((16, 128), jnp.float32),
            pltpu.VMEM_SHARED((NP, 8), jnp.float32),
            pltpu.SemaphoreType.DMA,
        ],
    )
    return f(src, dst, xpad, c1, z32)


def _tc_b_body(acc1_ref, w1f_ref, b1_ref, w2_ref, as2f_ref, ad2f_ref,
               h2t_ref, a2c_ref, c2_ref):
    i = pl.program_id(0)
    a = acc1_ref[...]
    den = a[0, :, 0:4] + a[1, :, 0:4] + 1e-16
    ssum = a[0, :, 4:8] + a[1, :, 4:8]
    out1 = ssum / den
    kcol = lax.broadcasted_iota(jnp.int32, (4, 64), 1) // 16
    hrow = lax.broadcasted_iota(jnp.int32, (4, 64), 0)
    p4 = (kcol == hrow).astype(jnp.float32)
    h1 = jnp.dot(out1, p4, preferred_element_type=jnp.float32,
                 precision=lax.Precision.HIGHEST)
    h1 = h1 * w1f_ref[...][None] + b1_ref[...][None]
    h1 = jnp.tanh(h1)
    h2 = jnp.dot(h1, w2_ref[...], preferred_element_type=jnp.float32,
                 precision=lax.Precision.HIGHEST)
    for p in range(8):
        h2t_ref[p] = h2[:, p * 16:(p + 1) * 16]
    kcol2 = lax.broadcasted_iota(jnp.int32, (128, 4), 0) // 32
    hrow2 = lax.broadcasted_iota(jnp.int32, (128, 4), 1)
    q4 = (kcol2 == hrow2).astype(jnp.float32)
    a2s = jnp.dot(h2 * as2f_ref[...][None], q4,
                  preferred_element_type=jnp.float32,
                  precision=lax.Precision.HIGHEST)
    a2d = jnp.dot(h2 * ad2f_ref[...][None], q4,
                  preferred_element_type=jnp.float32,
                  precision=lax.Precision.HIGHEST)
    a2c_ref[...] = jnp.concatenate([a2s, a2d], axis=1)
    pmax = jnp.max(a2s, axis=0)
    qmax = jnp.max(a2d, axis=0)
    rows = [jnp.full((1, 128), pmax[h], jnp.float32) for h in range(4)]
    rows += [jnp.full((1, 128), qmax[h], jnp.float32) for h in range(4)]
    cur = jnp.concatenate(rows, axis=0)

    @pl.when(i == 0)
    def _():
        c2_ref[...] = cur

    @pl.when(i > 0)
    def _():
        c2_ref[...] = jnp.maximum(c2_ref[...], cur)


def _tc_b(acc1, w1f, b1, w2, as2f, ad2f):
    grid = (NP // 1024,)
    return pl.pallas_call(
        _tc_b_body,
        grid=grid,
        in_specs=[
            pl.BlockSpec((NSC, 1024, 8), lambda i: (0, i, 0)),
            pl.BlockSpec((64,), lambda i: (0,)),
            pl.BlockSpec((64,), lambda i: (0,)),
            pl.BlockSpec((64, 128), lambda i: (0, 0)),
            pl.BlockSpec((128,), lambda i: (0,)),
            pl.BlockSpec((128,), lambda i: (0,)),
        ],
        out_specs=[
            pl.BlockSpec((8, 1024, 16), lambda i: (0, i, 0)),
            pl.BlockSpec((1024, 8), lambda i: (i, 0)),
            pl.BlockSpec((8, 128), lambda i: (0, 0)),
        ],
        out_shape=[
            jax.ShapeDtypeStruct((8, NP, 16), jnp.float32),
            jax.ShapeDtypeStruct((NP, 8), jnp.float32),
            jax.ShapeDtypeStruct((8, 128), jnp.float32),
        ],
    )(acc1, w1f, b1, w2, as2f, ad2f)


def _sc2_body(src_hbm, dst_hbm, att_hbm, c2_hbm, z_hbm,
              e2t_out, den_out,
              idxs, idxd, asr, adr, rows, e2b, c2v, acc, sem):
    c = lax.axis_index("c")
    s = lax.axis_index("s")
    pltpu.sync_copy(z_hbm.at[pl.ds(s * RPT, RPT), pl.ds(0, 8)],
                    acc.at[pl.ds(s * RPT, RPT)])
    pltpu.sync_copy(z_hbm.at[pl.ds(0, B), pl.ds(0, 8)], rows)
    pltpu.sync_copy(c2_hbm, c2v)
    plsc.subcore_barrier()
    m2 = []
    for h in range(4):
        mm = (c2v[h, pl.ds(0, 16)][0] + c2v[4 + h, pl.ds(0, 16)][0])
        m2.append(jnp.where(mm > 0, mm, 0.2 * mm))
    iota = lax.broadcasted_iota(jnp.int32, (16,), 0)

    @pl.loop(0, ROUNDS)
    def _round(r):
        base = c * PER_SC + (r * NT + s) * B
        pltpu.sync_copy(src_hbm.at[pl.ds(base, B)], idxs)
        pltpu.sync_copy(dst_hbm.at[pl.ds(base, B)], idxd)
        pltpu.async_copy(att_hbm.at[idxs], asr, sem).wait()
        pltpu.async_copy(att_hbm.at[idxd], adr, sem).wait()

        @pl.loop(0, B // 16)
        def _grp(g):
            ridx = g * 16 + iota
            for h in range(4):
                hv = jnp.full((16,), h, jnp.int32)
                av = plsc.load_gather(asr, [ridx, hv])
                bv = plsc.load_gather(adr, [ridx, hv + 4])
                a = av + bv
                a = jnp.where(a > 0, a, 0.2 * a)
                e = jnp.exp(a - m2[h])
                e2b[pl.ds(h * B + g * 16, 16)] = e
                plsc.store_scatter(rows, [ridx, hv], e)

        pltpu.sync_copy(rows, acc.at[idxd], add=True)
        for h in range(4):
            pltpu.sync_copy(e2b.at[pl.ds(h * B, B)],
                            e2t_out.at[h, pl.ds(base, B)])

    plsc.subcore_barrier()
    pltpu.sync_copy(acc.at[pl.ds(s * RPT, RPT)],
                    den_out.at[c, pl.ds(s * RPT, RPT)])


def _sc2(src, dst, att, c2, z32):
    mesh = plsc.VectorSubcoreMesh(core_axis_name="c", subcore_axis_name="s")
    f = pl.kernel(
        _sc2_body,
        out_type=(
            jax.ShapeDtypeStruct((4, EPP), jnp.float32),
            jax.ShapeDtypeStruct((NSC, NP, 8), jnp.float32),
        ),
        mesh=mesh,
        compiler_params=pltpu.CompilerParams(use_tc_tiling_on_sc=False, needs_layout_passes=False),
        scratch_types=[
            pltpu.VMEM((B,), jnp.int32),
            pltpu.VMEM((B,), jnp.int32),
            pltpu.VMEM((B, 8), jnp.float32),
            pltpu.VMEM((B, 8), jnp.float32),
            pltpu.VMEM((B, 8), jnp.float32),
            pltpu.VMEM((4 * B,), jnp.float32),
            pltpu.VMEM((8, 128), jnp.float32),
            pltpu.VMEM_SHARED((NP, 8), jnp.float32),
            pltpu.SemaphoreType.DMA,
        ],
    )
    return f(src, dst, att, c2, z32)


def _sc3_body(src_hbm, dst_hbm, e2t_hbm, h2t_hbm, z_hbm, msum_out,
              idxs, idxd0, idxd1, idx40, idx41, ev0, ev1, rows0, rows1,
              acc, semg0, semg1, sems0, sems1):
    c = lax.axis_index("c")
    s = lax.axis_index("s")
    idxd = (idxd0, idxd1)
    idx4 = (idx40, idx41)
    ev = (ev0, ev1)
    rows = (rows0, rows1)
    semg = (semg0, semg1)
    sems = (sems0, sems1)

    for p in range(8):
        pltpu.sync_copy(z_hbm.at[pl.ds(s * RPT, RPT)],
                        acc.at[pl.ds(s * RPT, RPT)])
        plsc.subcore_barrier()

        def load_prep(rr, b):
            base = c * PER_SC + (rr * NT + s) * B
            pltpu.sync_copy(src_hbm.at[pl.ds(base, B)], idxs)
            pltpu.sync_copy(dst_hbm.at[pl.ds(base, B)], idxd[b])
            pltpu.sync_copy(e2t_hbm.at[p // 2, pl.ds(base, B)], ev[b])

            @pl.loop(0, B // 16)
            def _gidx(g):
                iv = idxs[pl.ds(g * 16, 16)]
                idx4[b][pl.ds(g * 16, 16)] = iv + p * NP

            pltpu.async_copy(h2t_hbm.at[idx4[b]], rows[b], semg[b])

        def wait_gather(b):
            pltpu.make_async_copy(h2t_hbm.at[pl.ds(0, B)], rows[b],
                                  semg[b]).wait()

        def multiply(b):
            @pl.loop(0, B // 16)
            def _edge(g):
                evec = ev[b][pl.ds(g * 16, 16)]
                for i in range(16):
                    j = g * 16 + i
                    eb = jnp.full((16,), evec[i], jnp.float32)
                    rows[b][j, pl.ds(0, 16)] = rows[b][j, pl.ds(0, 16)] * eb

        load_prep(0, 0)
        load_prep(1, 1)

        @pl.loop(0, ROUNDS // 2)
        def _round(k):
            for b in range(2):
                rr = 2 * k + b
                wait_gather(b)
                multiply(b)
                pltpu.async_copy(rows[b], acc.at[idxd[b]], sems[b], add=True)

                @pl.when(rr + 2 < ROUNDS)
                def _():
                    pltpu.make_async_copy(rows[b], acc.at[idxd[b]],
                                          sems[b]).wait()
                    load_prep(rr + 2, b)

        pltpu.make_async_copy(rows[0], acc.at[idxd[0]], sems[0]).wait()
        pltpu.make_async_copy(rows[1], acc.at[idxd[1]], sems[1]).wait()
        plsc.subcore_barrier()
        pltpu.sync_copy(acc.at[pl.ds(s * RPT, RPT)],
                        msum_out.at[p, c, pl.ds(s * RPT, RPT)])
        plsc.subcore_barrier()


def _sc3(src, dst, e2t, h2t, z16):
    mesh = plsc.VectorSubcoreMesh(core_axis_name="c", subcore_axis_name="s")
    f = pl.kernel(
        _sc3_body,
        out_type=jax.ShapeDtypeStruct((8, NSC, NP, 16), jnp.float32),
        mesh=mesh,
        compiler_params=pltpu.CompilerParams(use_tc_tiling_on_sc=False, needs_layout_passes=False),
        scratch_types=[
            pltpu.VMEM((B,), jnp.int32),
            pltpu.VMEM((B,), jnp.int32),
            pltpu.VMEM((B,), jnp.int32),
            pltpu.VMEM((B,), jnp.int32),
            pltpu.VMEM((B,), jnp.int32),
            pltpu.VMEM((B,), jnp.float32),
            pltpu.VMEM((B,), jnp.float32),
            pltpu.VMEM((B, 16), jnp.float32),
            pltpu.VMEM((B, 16), jnp.float32),
            pltpu.VMEM_SHARED((NP, 16), jnp.float32),
            pltpu.SemaphoreType.DMA,
            pltpu.SemaphoreType.DMA,
            pltpu.SemaphoreType.DMA,
            pltpu.SemaphoreType.DMA,
        ],
    )
    return f(src, dst, e2t, h2t, z16)


def _tc_c_body(msum_ref, den_ref, b2_ref, out_ref):
    i = pl.program_id(0)
    m = msum_ref[...]
    den = den_ref[...]
    ms = m[:, 0] + m[:, 1]
    dn = den[0, :, 0:4] + den[1, :, 0:4] + 1e-16
    o = jnp.concatenate([ms[p] for p in range(8)], axis=1)
    kcol = lax.broadcasted_iota(jnp.int32, (4, 128), 1) // 32
    hrow = lax.broadcasted_iota(jnp.int32, (4, 128), 0)
    k4 = (kcol == hrow).astype(jnp.float32)
    dnrep = jnp.dot(dn, k4, preferred_element_type=jnp.float32,
                    precision=lax.Precision.HIGHEST)
    o = o / dnrep
    o = jnp.tanh(o + b2_ref[...][None])
    vid = i * 1024 + lax.broadcasted_iota(jnp.int32, (1024, 1), 0)
    o = jnp.where(vid < N, o, 0.0)
    psum = jnp.sum(o, axis=0, keepdims=True)

    @pl.when(i == 0)
    def _():
        out_ref[...] = psum

    @pl.when(i > 0)
    def _():
        out_ref[...] = out_ref[...] + psum

    @pl.when(i == NP // 1024 - 1)
    def _():
        out_ref[...] = out_ref[...] * (1.0 / N)


def _tc_c(msum, den2, b2):
    grid = (NP // 1024,)
    return pl.pallas_call(
        _tc_c_body,
        grid=grid,
        in_specs=[
            pl.BlockSpec((8, NSC, 1024, 16), lambda i: (0, 0, i, 0)),
            pl.BlockSpec((NSC, 1024, 8), lambda i: (0, i, 0)),
            pl.BlockSpec((128,), lambda i: (0,)),
        ],
        out_specs=pl.BlockSpec((1, 128), lambda i: (0, 0)),
        out_shape=jax.ShapeDtypeStruct((1, 128), jnp.float32),
    )(msum, den2, b2)


@jax.jit
def kernel(x, edge_index, W1, att_src1, att_dst1, b1, W2, att_src2,
           att_dst2, b2):
    ei = edge_index.astype(jnp.int32)
    loop = jnp.arange(N, dtype=jnp.int32)
    pad = jnp.full((EPP - EP,), TRASH, jnp.int32)
    src = jnp.concatenate([ei[0], loop, pad])
    dst = jnp.concatenate([ei[1], loop, pad])
    xflat = x[:, 0]
    xpad = jnp.pad(xflat, (0, NP - N), mode="edge")
    x2d = xpad.reshape(392, 128)
    z32 = jnp.zeros((NP, 32), jnp.float32)
    z16 = jnp.zeros((NP, 16), jnp.float32)

    c1 = _tc_a(x2d, W1.reshape(4, 16), att_src1[0], att_dst1[0])
    acc1 = _sc1(src, dst, xpad, c1, z32)
    h2p, a2c, c2 = _tc_b(acc1, W1[0], b1, W2,
                         att_src2.reshape(128), att_dst2.reshape(128))
    h2t = h2p.reshape(NP * 8, 16)
    e2t, den2 = _sc2(src, dst, a2c, c2, z32)
    msum = _sc3(src, dst, e2t, h2t, z16)
    return _tc_c(msum, den2, b2)


# speedup vs baseline: 131.6458x; 1.2990x over previous
import functools

import jax
import jax.numpy as jnp
from jax import lax
from jax.experimental import pallas as pl
from jax.experimental.pallas import tpu as pltpu
from jax.experimental.pallas import tpu_sc as plsc

N = 50000
E = 800000
EP = N + E
NP = 50176
N4P = NP * 4
TRASH = N
B = 1024
NSC = 2
NT = 16
ROUNDS = -(-EP // (NSC * NT * B))
EPP = ROUNDS * NSC * NT * B
PER_SC = EPP // NSC
RPT = NP // NT


def _tc_a_body(x2d_ref, w1r_ref, as1_ref, ad1_ref, c1_ref):
    x2d = x2d_ref[...]
    xmax = jnp.max(x2d)
    xmin = jnp.min(x2d)
    w1r = w1r_ref[...]
    s1 = jnp.sum(w1r * as1_ref[...], axis=1)
    d1 = jnp.sum(w1r * ad1_ref[...], axis=1)
    p1 = jnp.maximum(xmax * s1, xmin * s1)
    q1 = jnp.maximum(xmax * d1, xmin * d1)
    m = p1 + q1
    m1 = jnp.where(m > 0, m, 0.2 * m)
    rows = []
    for h in range(4):
        rows.append(jnp.full((1, 128), s1[h], jnp.float32))
    for h in range(4):
        rows.append(jnp.full((1, 128), d1[h], jnp.float32))
    for h in range(4):
        rows.append(jnp.full((1, 128), m1[h], jnp.float32))
    rows.append(jnp.zeros((4, 128), jnp.float32))
    c1_ref[...] = jnp.concatenate(rows, axis=0)


def _tc_a(x2d, w1r, as1r, ad1r):
    return pl.pallas_call(
        _tc_a_body,
        out_shape=jax.ShapeDtypeStruct((16, 128), jnp.float32),
    )(x2d, w1r, as1r, ad1r)


def _sc1_body(src_hbm, dst_hbm, xpad_hbm, c1_hbm, z_hbm, acc_out,
              idxs, idxd, xs, xd, rows, c1v, acc, sem):
    c = lax.axis_index("c")
    s = lax.axis_index("s")
    pltpu.sync_copy(z_hbm.at[pl.ds(s * RPT, RPT), pl.ds(0, 8)],
                    acc.at[pl.ds(s * RPT, RPT)])
    pltpu.sync_copy(c1_hbm, c1v)
    plsc.subcore_barrier()
    s1 = [c1v[h, pl.ds(0, 16)][0] for h in range(4)]
    d1 = [c1v[4 + h, pl.ds(0, 16)][0] for h in range(4)]
    m1 = [c1v[8 + h, pl.ds(0, 16)][0] for h in range(4)]
    iota = lax.broadcasted_iota(jnp.int32, (16,), 0)

    @pl.loop(0, ROUNDS)
    def _round(r):
        base = c * PER_SC + (r * NT + s) * B
        pltpu.sync_copy(src_hbm.at[pl.ds(base, B)], idxs)
        pltpu.sync_copy(dst_hbm.at[pl.ds(base, B)], idxd)
        pltpu.async_copy(xpad_hbm.at[idxs], xs, sem).wait()
        pltpu.async_copy(xpad_hbm.at[idxd], xd, sem).wait()

        @pl.loop(0, B // 16)
        def _grp(g):
            vs = xs[pl.ds(g * 16, 16)]
            vd = xd[pl.ds(g * 16, 16)]
            ridx = g * 16 + iota
            for h in range(4):
                a = vs * s1[h] + vd * d1[h]
                a = jnp.where(a > 0, a, 0.2 * a)
                e = jnp.exp(a - m1[h])
                hv = jnp.full((16,), h, jnp.int32)
                plsc.store_scatter(rows, [ridx, hv], e)
                plsc.store_scatter(rows, [ridx, hv + 4], e * vs)

        pltpu.sync_copy(rows, acc.at[idxd], add=True)

    plsc.subcore_barrier()
    pltpu.sync_copy(acc.at[pl.ds(s * RPT, RPT)],
                    acc_out.at[c, pl.ds(s * RPT, RPT)])


def _sc1(src, dst, xpad, c1, z32):
    mesh = plsc.VectorSubcoreMesh(core_axis_name="c", subcore_axis_name="s")
    f = pl.kernel(
        _sc1_body,
        out_type=jax.ShapeDtypeStruct((NSC, NP, 8), jnp.float32),
        mesh=mesh,
        compiler_params=pltpu.CompilerParams(use_tc_tiling_on_sc=False, needs_layout_passes=False),
        scratch_types=[
            pltpu.VMEM((B,), jnp.int32),
            pltpu.VMEM((B,), jnp.int32),
            pltpu.VMEM((B,), jnp.float32),
            pltpu.VMEM((B,), jnp.float32),
            pltpu.VMEM((B, 8), jnp.float32),
            pltpu.VMEM((16, 128), jnp.float32),
            pltpu.VMEM_SHARED((NP, 8), jnp.float32),
            pltpu.SemaphoreType.DMA,
        ],
    )
    return f(src, dst, xpad, c1, z32)


def _tc_b_body(acc1_ref, w1f_ref, b1_ref, w2_ref, as2f_ref, ad2f_ref,
               h2t_ref, a2c_ref, c2_ref):
    i = pl.program_id(0)
    a = acc1_ref[...]
    den = a[0, :, 0:4] + a[1, :, 0:4] + 1e-16
    ssum = a[0, :, 4:8] + a[1, :, 4:8]
    out1 = ssum / den
    kcol = lax.broadcasted_iota(jnp.int32, (4, 64), 1) // 16
    hrow = lax.broadcasted_iota(jnp.int32, (4, 64), 0)
    p4 = (kcol == hrow).astype(jnp.float32)
    h1 = jnp.dot(out1, p4, preferred_element_type=jnp.float32,
                 precision=lax.Precision.HIGHEST)
    h1 = h1 * w1f_ref[...][None] + b1_ref[...][None]
    h1 = jnp.tanh(h1)
    h2 = jnp.dot(h1, w2_ref[...], preferred_element_type=jnp.float32,
                 precision=lax.Precision.HIGHEST)
    for p in range(4):
        h2t_ref[p] = h2[:, p * 32:(p + 1) * 32].astype(jnp.bfloat16)
    kcol2 = lax.broadcasted_iota(jnp.int32, (128, 4), 0) // 32
    hrow2 = lax.broadcasted_iota(jnp.int32, (128, 4), 1)
    q4 = (kcol2 == hrow2).astype(jnp.float32)
    a2s = jnp.dot(h2 * as2f_ref[...][None], q4,
                  preferred_element_type=jnp.float32,
                  precision=lax.Precision.HIGHEST)
    a2d = jnp.dot(h2 * ad2f_ref[...][None], q4,
                  preferred_element_type=jnp.float32,
                  precision=lax.Precision.HIGHEST)
    a2c_ref[...] = jnp.concatenate([a2s, a2d], axis=1)
    pmax = jnp.max(a2s, axis=0)
    qmax = jnp.max(a2d, axis=0)
    rows = [jnp.full((1, 128), pmax[h], jnp.float32) for h in range(4)]
    rows += [jnp.full((1, 128), qmax[h], jnp.float32) for h in range(4)]
    cur = jnp.concatenate(rows, axis=0)

    @pl.when(i == 0)
    def _():
        c2_ref[...] = cur

    @pl.when(i > 0)
    def _():
        c2_ref[...] = jnp.maximum(c2_ref[...], cur)


def _tc_b(acc1, w1f, b1, w2, as2f, ad2f):
    grid = (NP // 1024,)
    return pl.pallas_call(
        _tc_b_body,
        grid=grid,
        in_specs=[
            pl.BlockSpec((NSC, 1024, 8), lambda i: (0, i, 0)),
            pl.BlockSpec((64,), lambda i: (0,)),
            pl.BlockSpec((64,), lambda i: (0,)),
            pl.BlockSpec((64, 128), lambda i: (0, 0)),
            pl.BlockSpec((128,), lambda i: (0,)),
            pl.BlockSpec((128,), lambda i: (0,)),
        ],
        out_specs=[
            pl.BlockSpec((4, 1024, 32), lambda i: (0, i, 0)),
            pl.BlockSpec((1024, 8), lambda i: (i, 0)),
            pl.BlockSpec((8, 128), lambda i: (0, 0)),
        ],
        out_shape=[
            jax.ShapeDtypeStruct((4, NP, 32), jnp.bfloat16),
            jax.ShapeDtypeStruct((NP, 8), jnp.float32),
            jax.ShapeDtypeStruct((8, 128), jnp.float32),
        ],
    )(acc1, w1f, b1, w2, as2f, ad2f)


def _sc2_body(src_hbm, dst_hbm, att_hbm, c2_hbm, z_hbm,
              e2t_out, den_out,
              idxs, idxd, asr, adr, rows, e2b, c2v, acc, sem):
    c = lax.axis_index("c")
    s = lax.axis_index("s")
    pltpu.sync_copy(z_hbm.at[pl.ds(s * RPT, RPT), pl.ds(0, 8)],
                    acc.at[pl.ds(s * RPT, RPT)])
    pltpu.sync_copy(z_hbm.at[pl.ds(0, B), pl.ds(0, 8)], rows)
    pltpu.sync_copy(c2_hbm, c2v)
    plsc.subcore_barrier()
    m2 = []
    for h in range(4):
        mm = (c2v[h, pl.ds(0, 16)][0] + c2v[4 + h, pl.ds(0, 16)][0])
        m2.append(jnp.where(mm > 0, mm, 0.2 * mm))
    iota = lax.broadcasted_iota(jnp.int32, (16,), 0)

    @pl.loop(0, ROUNDS)
    def _round(r):
        base = c * PER_SC + (r * NT + s) * B
        pltpu.sync_copy(src_hbm.at[pl.ds(base, B)], idxs)
        pltpu.sync_copy(dst_hbm.at[pl.ds(base, B)], idxd)
        pltpu.async_copy(att_hbm.at[idxs], asr, sem).wait()
        pltpu.async_copy(att_hbm.at[idxd], adr, sem).wait()

        @pl.loop(0, B // 16)
        def _grp(g):
            ridx = g * 16 + iota
            for h in range(4):
                hv = jnp.full((16,), h, jnp.int32)
                av = plsc.load_gather(asr, [ridx, hv])
                bv = plsc.load_gather(adr, [ridx, hv + 4])
                a = av + bv
                a = jnp.where(a > 0, a, 0.2 * a)
                e = jnp.exp(a - m2[h])
                e2b[pl.ds(h * B + g * 16, 16)] = e
                plsc.store_scatter(rows, [ridx, hv], e)

        pltpu.sync_copy(rows, acc.at[idxd], add=True)
        for h in range(4):
            pltpu.sync_copy(e2b.at[pl.ds(h * B, B)],
                            e2t_out.at[h, pl.ds(base, B)])

    plsc.subcore_barrier()
    pltpu.sync_copy(acc.at[pl.ds(s * RPT, RPT)],
                    den_out.at[c, pl.ds(s * RPT, RPT)])


def _sc2(src, dst, att, c2, z32):
    mesh = plsc.VectorSubcoreMesh(core_axis_name="c", subcore_axis_name="s")
    f = pl.kernel(
        _sc2_body,
        out_type=(
            jax.ShapeDtypeStruct((4, EPP), jnp.float32),
            jax.ShapeDtypeStruct((NSC, NP, 8), jnp.float32),
        ),
        mesh=mesh,
        compiler_params=pltpu.CompilerParams(use_tc_tiling_on_sc=False, needs_layout_passes=False),
        scratch_types=[
            pltpu.VMEM((B,), jnp.int32),
            pltpu.VMEM((B,), jnp.int32),
            pltpu.VMEM((B, 8), jnp.float32),
            pltpu.VMEM((B, 8), jnp.float32),
            pltpu.VMEM((B, 8), jnp.float32),
            pltpu.VMEM((4 * B,), jnp.float32),
            pltpu.VMEM((8, 128), jnp.float32),
            pltpu.VMEM_SHARED((NP, 8), jnp.float32),
            pltpu.SemaphoreType.DMA,
        ],
    )
    return f(src, dst, att, c2, z32)


def _sc3_body(src_hbm, dst_hbm, e2t_hbm, h2t_hbm, z_hbm, msum_out,
              idxs, idxd0, idxd1, idx40, idx41, ev0, ev1, rows0, rows1,
              acc, semg0, semg1, sems0, sems1):
    c = lax.axis_index("c")
    s = lax.axis_index("s")
    idxd = (idxd0, idxd1)
    idx4 = (idx40, idx41)
    ev = (ev0, ev1)
    rows = (rows0, rows1)
    semg = (semg0, semg1)
    sems = (sems0, sems1)

    for p in range(4):
        pltpu.sync_copy(z_hbm.at[pl.ds(s * RPT, RPT)],
                        acc.at[pl.ds(s * RPT, RPT)])
        plsc.subcore_barrier()

        def load_prep(rr, b):
            base = c * PER_SC + (rr * NT + s) * B
            pltpu.sync_copy(src_hbm.at[pl.ds(base, B)], idxs)
            pltpu.sync_copy(dst_hbm.at[pl.ds(base, B)], idxd[b])
            pltpu.sync_copy(e2t_hbm.at[p, pl.ds(base, B)], ev[b])

            @pl.loop(0, B // 16)
            def _gidx(g):
                iv = idxs[pl.ds(g * 16, 16)]
                idx4[b][pl.ds(g * 16, 16)] = iv + p * NP

            pltpu.async_copy(h2t_hbm.at[idx4[b]], rows[b], semg[b])

        def wait_gather(b):
            pltpu.make_async_copy(h2t_hbm.at[pl.ds(0, B)], rows[b],
                                  semg[b]).wait()

        def multiply(b):
            @pl.loop(0, B // 16)
            def _edge(g):
                evec = ev[b][pl.ds(g * 16, 16)]
                for i in range(16):
                    j = g * 16 + i
                    ef = jnp.full((16,), evec[i], jnp.float32)
                    eb = plsc.pack(ef, ef, format=plsc.PackFormat.INTERLEAVED)
                    rows[b][j, pl.ds(0, 32)] = rows[b][j, pl.ds(0, 32)] * eb

        load_prep(0, 0)
        load_prep(1, 1)

        @pl.loop(0, ROUNDS // 2)
        def _round(k):
            for b in range(2):
                rr = 2 * k + b
                wait_gather(b)
                multiply(b)
                pltpu.async_copy(rows[b], acc.at[idxd[b]], sems[b], add=True)

                @pl.when(rr + 2 < ROUNDS)
                def _():
                    pltpu.make_async_copy(rows[b], acc.at[idxd[b]],
                                          sems[b]).wait()
                    load_prep(rr + 2, b)

        pltpu.make_async_copy(rows[0], acc.at[idxd[0]], sems[0]).wait()
        pltpu.make_async_copy(rows[1], acc.at[idxd[1]], sems[1]).wait()
        plsc.subcore_barrier()
        pltpu.sync_copy(acc.at[pl.ds(s * RPT, RPT)],
                        msum_out.at[p, c, pl.ds(s * RPT, RPT)])
        plsc.subcore_barrier()


def _sc3(src, dst, e2t, h2t, zb):
    mesh = plsc.VectorSubcoreMesh(core_axis_name="c", subcore_axis_name="s")
    f = pl.kernel(
        _sc3_body,
        out_type=jax.ShapeDtypeStruct((4, NSC, NP, 32), jnp.bfloat16),
        mesh=mesh,
        compiler_params=pltpu.CompilerParams(use_tc_tiling_on_sc=False, needs_layout_passes=False),
        scratch_types=[
            pltpu.VMEM((B,), jnp.int32),
            pltpu.VMEM((B,), jnp.int32),
            pltpu.VMEM((B,), jnp.int32),
            pltpu.VMEM((B,), jnp.int32),
            pltpu.VMEM((B,), jnp.int32),
            pltpu.VMEM((B,), jnp.float32),
            pltpu.VMEM((B,), jnp.float32),
            pltpu.VMEM((B, 32), jnp.bfloat16),
            pltpu.VMEM((B, 32), jnp.bfloat16),
            pltpu.VMEM_SHARED((NP, 32), jnp.bfloat16),
            pltpu.SemaphoreType.DMA,
            pltpu.SemaphoreType.DMA,
            pltpu.SemaphoreType.DMA,
            pltpu.SemaphoreType.DMA,
        ],
    )
    return f(src, dst, e2t, h2t, zb)


def _tc_c_body(msum_ref, den_ref, b2_ref, out_ref):
    i = pl.program_id(0)
    m = msum_ref[...].astype(jnp.float32)
    den = den_ref[...]
    ms = m[:, 0] + m[:, 1]
    dn = den[0, :, 0:4] + den[1, :, 0:4] + 1e-16
    o = jnp.concatenate([ms[p] for p in range(4)], axis=1)
    kcol = lax.broadcasted_iota(jnp.int32, (4, 128), 1) // 32
    hrow = lax.broadcasted_iota(jnp.int32, (4, 128), 0)
    k4 = (kcol == hrow).astype(jnp.float32)
    dnrep = jnp.dot(dn, k4, preferred_element_type=jnp.float32,
                    precision=lax.Precision.HIGHEST)
    o = o / dnrep
    o = jnp.tanh(o + b2_ref[...][None])
    vid = i * 1024 + lax.broadcasted_iota(jnp.int32, (1024, 1), 0)
    o = jnp.where(vid < N, o, 0.0)
    psum = jnp.sum(o, axis=0, keepdims=True)

    @pl.when(i == 0)
    def _():
        out_ref[...] = psum

    @pl.when(i > 0)
    def _():
        out_ref[...] = out_ref[...] + psum

    @pl.when(i == NP // 1024 - 1)
    def _():
        out_ref[...] = out_ref[...] * (1.0 / N)


def _tc_c(msum, den2, b2):
    grid = (NP // 1024,)
    return pl.pallas_call(
        _tc_c_body,
        grid=grid,
        in_specs=[
            pl.BlockSpec((4, NSC, 1024, 32), lambda i: (0, 0, i, 0)),
            pl.BlockSpec((NSC, 1024, 8), lambda i: (0, i, 0)),
            pl.BlockSpec((128,), lambda i: (0,)),
        ],
        out_specs=pl.BlockSpec((1, 128), lambda i: (0, 0)),
        out_shape=jax.ShapeDtypeStruct((1, 128), jnp.float32),
    )(msum, den2, b2)


@jax.jit
def kernel(x, edge_index, W1, att_src1, att_dst1, b1, W2, att_src2,
           att_dst2, b2):
    ei = edge_index.astype(jnp.int32)
    loop = jnp.arange(N, dtype=jnp.int32)
    pad = jnp.full((EPP - EP,), TRASH, jnp.int32)
    src = jnp.concatenate([ei[0], loop, pad])
    dst = jnp.concatenate([ei[1], loop, pad])
    xflat = x[:, 0]
    xpad = jnp.pad(xflat, (0, NP - N), mode="edge")
    x2d = xpad.reshape(392, 128)
    z32 = jnp.zeros((NP, 32), jnp.float32)
    zb = jnp.zeros((NP, 32), jnp.bfloat16)

    c1 = _tc_a(x2d, W1.reshape(4, 16), att_src1[0], att_dst1[0])
    acc1 = _sc1(src, dst, xpad, c1, z32)
    h2p, a2c, c2 = _tc_b(acc1, W1[0], b1, W2,
                         att_src2.reshape(128), att_dst2.reshape(128))
    h2t = h2p.reshape(NP * 4, 32)
    e2t, den2 = _sc2(src, dst, a2c, c2, z32)
    msum = _sc3(src, dst, e2t, h2t, zb)
    return _tc_c(msum, den2, b2)


# speedup vs baseline: 140.3550x; 1.0662x over previous
import functools

import jax
import jax.numpy as jnp
from jax import lax
from jax.experimental import pallas as pl
from jax.experimental.pallas import tpu as pltpu
from jax.experimental.pallas import tpu_sc as plsc

N = 50000
E = 800000
EP = N + E
NP = 50176
N4P = NP * 4
TRASH = N
B = 1024
NSC = 2
NT = 16
ROUNDS = -(-EP // (NSC * NT * B))
EPP = ROUNDS * NSC * NT * B
PER_SC = EPP // NSC
RPT = NP // NT


def _tc_a_body(x2d_ref, w1r_ref, as1_ref, ad1_ref, c1_ref):
    x2d = x2d_ref[...]
    xmax = jnp.max(x2d)
    xmin = jnp.min(x2d)
    w1r = w1r_ref[...]
    s1 = jnp.sum(w1r * as1_ref[...], axis=1)
    d1 = jnp.sum(w1r * ad1_ref[...], axis=1)
    p1 = jnp.maximum(xmax * s1, xmin * s1)
    q1 = jnp.maximum(xmax * d1, xmin * d1)
    m = p1 + q1
    m1 = jnp.where(m > 0, m, 0.2 * m)
    rows = []
    for h in range(4):
        rows.append(jnp.full((1, 128), s1[h], jnp.float32))
    for h in range(4):
        rows.append(jnp.full((1, 128), d1[h], jnp.float32))
    for h in range(4):
        rows.append(jnp.full((1, 128), m1[h], jnp.float32))
    rows.append(jnp.zeros((4, 128), jnp.float32))
    c1_ref[...] = jnp.concatenate(rows, axis=0)


def _tc_a(x2d, w1r, as1r, ad1r):
    return pl.pallas_call(
        _tc_a_body,
        out_shape=jax.ShapeDtypeStruct((16, 128), jnp.float32),
    )(x2d, w1r, as1r, ad1r)


def _sc1_body(src_hbm, dst_hbm, xpad_hbm, c1_hbm, z_hbm, acc_out,
              idxs, idxd, xs, xd, rows, c1v, acc, sem):
    c = lax.axis_index("c")
    s = lax.axis_index("s")
    pltpu.sync_copy(z_hbm.at[pl.ds(s * RPT, RPT), pl.ds(0, 8)],
                    acc.at[pl.ds(s * RPT, RPT)])
    pltpu.sync_copy(c1_hbm, c1v)
    plsc.subcore_barrier()
    s1 = [c1v[h, pl.ds(0, 16)][0] for h in range(4)]
    d1 = [c1v[4 + h, pl.ds(0, 16)][0] for h in range(4)]
    m1 = [c1v[8 + h, pl.ds(0, 16)][0] for h in range(4)]
    iota = lax.broadcasted_iota(jnp.int32, (16,), 0)

    @pl.loop(0, ROUNDS)
    def _round(r):
        base = c * PER_SC + (r * NT + s) * B
        pltpu.sync_copy(src_hbm.at[pl.ds(base, B)], idxs)
        pltpu.sync_copy(dst_hbm.at[pl.ds(base, B)], idxd)
        pltpu.async_copy(xpad_hbm.at[idxs], xs, sem).wait()
        pltpu.async_copy(xpad_hbm.at[idxd], xd, sem).wait()

        @pl.loop(0, B // 16)
        def _grp(g):
            vs = xs[pl.ds(g * 16, 16)]
            vd = xd[pl.ds(g * 16, 16)]
            ridx = g * 16 + iota
            for h in range(4):
                a = vs * s1[h] + vd * d1[h]
                a = jnp.where(a > 0, a, 0.2 * a)
                e = jnp.exp(a - m1[h])
                hv = jnp.full((16,), h, jnp.int32)
                plsc.store_scatter(rows, [ridx, hv], e)
                plsc.store_scatter(rows, [ridx, hv + 4], e * vs)

        pltpu.sync_copy(rows, acc.at[idxd], add=True)

    plsc.subcore_barrier()
    pltpu.sync_copy(acc.at[pl.ds(s * RPT, RPT)],
                    acc_out.at[c, pl.ds(s * RPT, RPT)])


def _sc1(src, dst, xpad, c1, z32):
    mesh = plsc.VectorSubcoreMesh(core_axis_name="c", subcore_axis_name="s")
    f = pl.kernel(
        _sc1_body,
        out_type=jax.ShapeDtypeStruct((NSC, NP, 8), jnp.float32),
        mesh=mesh,
        compiler_params=pltpu.CompilerParams(use_tc_tiling_on_sc=False, needs_layout_passes=False),
        scratch_types=[
            pltpu.VMEM((B,), jnp.int32),
            pltpu.VMEM((B,), jnp.int32),
            pltpu.VMEM((B,), jnp.float32),
            pltpu.VMEM((B,), jnp.float32),
            pltpu.VMEM((B, 8), jnp.float32),
            pltpu.VMEM((16, 128), jnp.float32),
            pltpu.VMEM_SHARED((NP, 8), jnp.float32),
            pltpu.SemaphoreType.DMA,
        ],
    )
    return f(src, dst, xpad, c1, z32)


def _tc_b_body(acc1_ref, w1f_ref, b1_ref, w2_ref, as2f_ref, ad2f_ref,
               h2t_ref, a2c_ref, c2_ref):
    i = pl.program_id(0)
    a = acc1_ref[...]
    den = a[0, :, 0:4] + a[1, :, 0:4] + 1e-16
    ssum = a[0, :, 4:8] + a[1, :, 4:8]
    out1 = ssum / den
    kcol = lax.broadcasted_iota(jnp.int32, (4, 64), 1) // 16
    hrow = lax.broadcasted_iota(jnp.int32, (4, 64), 0)
    p4 = (kcol == hrow).astype(jnp.float32)
    h1 = jnp.dot(out1, p4, preferred_element_type=jnp.float32,
                 precision=lax.Precision.HIGHEST)
    h1 = h1 * w1f_ref[...][None] + b1_ref[...][None]
    h1 = jnp.tanh(h1)
    h2 = jnp.dot(h1, w2_ref[...], preferred_element_type=jnp.float32,
                 precision=lax.Precision.HIGHEST)
    for p in range(4):
        h2t_ref[p] = h2[:, p * 32:(p + 1) * 32].astype(jnp.bfloat16)
    kcol2 = lax.broadcasted_iota(jnp.int32, (128, 4), 0) // 32
    hrow2 = lax.broadcasted_iota(jnp.int32, (128, 4), 1)
    q4 = (kcol2 == hrow2).astype(jnp.float32)
    a2s = jnp.dot(h2 * as2f_ref[...][None], q4,
                  preferred_element_type=jnp.float32,
                  precision=lax.Precision.HIGHEST)
    a2d = jnp.dot(h2 * ad2f_ref[...][None], q4,
                  preferred_element_type=jnp.float32,
                  precision=lax.Precision.HIGHEST)
    a2c_ref[...] = jnp.concatenate([a2s, a2d], axis=1)
    pmax = jnp.max(a2s, axis=0)
    qmax = jnp.max(a2d, axis=0)
    rows = [jnp.full((1, 128), pmax[h], jnp.float32) for h in range(4)]
    rows += [jnp.full((1, 128), qmax[h], jnp.float32) for h in range(4)]
    cur = jnp.concatenate(rows, axis=0)

    @pl.when(i == 0)
    def _():
        c2_ref[...] = cur

    @pl.when(i > 0)
    def _():
        c2_ref[...] = jnp.maximum(c2_ref[...], cur)


def _tc_b(acc1, w1f, b1, w2, as2f, ad2f):
    grid = (NP // 1024,)
    return pl.pallas_call(
        _tc_b_body,
        grid=grid,
        in_specs=[
            pl.BlockSpec((NSC, 1024, 8), lambda i: (0, i, 0)),
            pl.BlockSpec((64,), lambda i: (0,)),
            pl.BlockSpec((64,), lambda i: (0,)),
            pl.BlockSpec((64, 128), lambda i: (0, 0)),
            pl.BlockSpec((128,), lambda i: (0,)),
            pl.BlockSpec((128,), lambda i: (0,)),
        ],
        out_specs=[
            pl.BlockSpec((4, 1024, 32), lambda i: (0, i, 0)),
            pl.BlockSpec((1024, 8), lambda i: (i, 0)),
            pl.BlockSpec((8, 128), lambda i: (0, 0)),
        ],
        out_shape=[
            jax.ShapeDtypeStruct((4, NP, 32), jnp.bfloat16),
            jax.ShapeDtypeStruct((NP, 8), jnp.float32),
            jax.ShapeDtypeStruct((8, 128), jnp.float32),
        ],
    )(acc1, w1f, b1, w2, as2f, ad2f)


def _sc2_body(src_hbm, dst_hbm, att_hbm, c2_hbm, z_hbm,
              e2t_out, den_out,
              idxs, idxd0, idxd1, asr0, asr1, adr0, adr1,
              rows0, rows1, e2b0, e2b1, c2v, acc,
              semg0, semg1, sems0, sems1, semw0, semw1):
    c = lax.axis_index("c")
    s = lax.axis_index("s")
    idxd = (idxd0, idxd1)
    asr = (asr0, asr1)
    adr = (adr0, adr1)
    rows = (rows0, rows1)
    e2b = (e2b0, e2b1)
    semg = (semg0, semg1)
    sems = (sems0, sems1)
    semw = (semw0, semw1)
    pltpu.sync_copy(z_hbm.at[pl.ds(s * RPT, RPT), pl.ds(0, 8)],
                    acc.at[pl.ds(s * RPT, RPT)])
    pltpu.sync_copy(z_hbm.at[pl.ds(0, B), pl.ds(0, 8)], rows0)
    pltpu.sync_copy(z_hbm.at[pl.ds(0, B), pl.ds(0, 8)], rows1)
    pltpu.sync_copy(c2_hbm, c2v)
    plsc.subcore_barrier()
    m2 = []
    for h in range(4):
        mm = (c2v[h, pl.ds(0, 16)][0] + c2v[4 + h, pl.ds(0, 16)][0])
        m2.append(jnp.where(mm > 0, mm, 0.2 * mm))
    iota = lax.broadcasted_iota(jnp.int32, (16,), 0)

    def load_prep(rr, b):
        base = c * PER_SC + (rr * NT + s) * B
        pltpu.sync_copy(src_hbm.at[pl.ds(base, B)], idxs)
        pltpu.sync_copy(dst_hbm.at[pl.ds(base, B)], idxd[b])
        pltpu.async_copy(att_hbm.at[idxs], asr[b], semg[b])
        pltpu.async_copy(att_hbm.at[idxd[b]], adr[b], semg[b])

    def compute(rr, b):
        base = c * PER_SC + (rr * NT + s) * B
        pltpu.make_async_copy(att_hbm.at[pl.ds(0, B)], asr[b], semg[b]).wait()
        pltpu.make_async_copy(att_hbm.at[pl.ds(0, B)], adr[b], semg[b]).wait()

        @pl.loop(0, B // 16)
        def _grp(g):
            ridx = g * 16 + iota
            for h in range(4):
                hv = jnp.full((16,), h, jnp.int32)
                av = plsc.load_gather(asr[b], [ridx, hv])
                bv = plsc.load_gather(adr[b], [ridx, hv + 4])
                a = av + bv
                a = jnp.where(a > 0, a, 0.2 * a)
                e = jnp.exp(a - m2[h])
                e2b[b][pl.ds(h * B + g * 16, 16)] = e
                plsc.store_scatter(rows[b], [ridx, hv], e)

        pltpu.async_copy(rows[b], acc.at[idxd[b]], sems[b], add=True)
        for h in range(4):
            pltpu.async_copy(e2b[b].at[pl.ds(h * B, B)],
                             e2t_out.at[h, pl.ds(base, B)], semw[b])

    def drain(rr, b):
        base = c * PER_SC + (rr * NT + s) * B
        pltpu.make_async_copy(rows[b], acc.at[idxd[b]], sems[b]).wait()
        for h in range(4):
            pltpu.make_async_copy(e2b[b].at[pl.ds(h * B, B)],
                                  e2t_out.at[h, pl.ds(base, B)],
                                  semw[b]).wait()

    load_prep(0, 0)
    load_prep(1, 1)

    @pl.loop(0, ROUNDS // 2)
    def _round(k):
        for b in range(2):
            rr = 2 * k + b
            compute(rr, b)

            @pl.when(rr + 2 < ROUNDS)
            def _():
                drain(rr, b)
                load_prep(rr + 2, b)

    drain(ROUNDS - 2, 0)
    drain(ROUNDS - 1, 1)
    plsc.subcore_barrier()
    pltpu.sync_copy(acc.at[pl.ds(s * RPT, RPT)],
                    den_out.at[c, pl.ds(s * RPT, RPT)])


def _sc2(src, dst, att, c2, z32):
    mesh = plsc.VectorSubcoreMesh(core_axis_name="c", subcore_axis_name="s")
    f = pl.kernel(
        _sc2_body,
        out_type=(
            jax.ShapeDtypeStruct((4, EPP), jnp.float32),
            jax.ShapeDtypeStruct((NSC, NP, 8), jnp.float32),
        ),
        mesh=mesh,
        compiler_params=pltpu.CompilerParams(use_tc_tiling_on_sc=False, needs_layout_passes=False),
        scratch_types=[
            pltpu.VMEM((B,), jnp.int32),
            pltpu.VMEM((B,), jnp.int32),
            pltpu.VMEM((B,), jnp.int32),
            pltpu.VMEM((B, 8), jnp.float32),
            pltpu.VMEM((B, 8), jnp.float32),
            pltpu.VMEM((B, 8), jnp.float32),
            pltpu.VMEM((B, 8), jnp.float32),
            pltpu.VMEM((B, 8), jnp.float32),
            pltpu.VMEM((B, 8), jnp.float32),
            pltpu.VMEM((4 * B,), jnp.float32),
            pltpu.VMEM((4 * B,), jnp.float32),
            pltpu.VMEM((8, 128), jnp.float32),
            pltpu.VMEM_SHARED((NP, 8), jnp.float32),
            pltpu.SemaphoreType.DMA,
            pltpu.SemaphoreType.DMA,
            pltpu.SemaphoreType.DMA,
            pltpu.SemaphoreType.DMA,
            pltpu.SemaphoreType.DMA,
            pltpu.SemaphoreType.DMA,
        ],
    )
    return f(src, dst, att, c2, z32)


def _sc3_body(src_hbm, dst_hbm, e2t_hbm, h2t_hbm, z_hbm, msum_out,
              idxs, idxd0, idxd1, idx40, idx41, ev0, ev1, rows0, rows1,
              acc, semg0, semg1, sems0, sems1):
    c = lax.axis_index("c")
    s = lax.axis_index("s")
    idxd = (idxd0, idxd1)
    idx4 = (idx40, idx41)
    ev = (ev0, ev1)
    rows = (rows0, rows1)
    semg = (semg0, semg1)
    sems = (sems0, sems1)

    for p in range(4):
        pltpu.sync_copy(z_hbm.at[pl.ds(s * RPT, RPT)],
                        acc.at[pl.ds(s * RPT, RPT)])
        plsc.subcore_barrier()

        def load_prep(rr, b):
            base = c * PER_SC + (rr * NT + s) * B
            pltpu.sync_copy(src_hbm.at[pl.ds(base, B)], idxs)
            pltpu.sync_copy(dst_hbm.at[pl.ds(base, B)], idxd[b])
            pltpu.sync_copy(e2t_hbm.at[p, pl.ds(base, B)], ev[b])

            @pl.loop(0, B // 16)
            def _gidx(g):
                iv = idxs[pl.ds(g * 16, 16)]
                idx4[b][pl.ds(g * 16, 16)] = iv + p * NP

            pltpu.async_copy(h2t_hbm.at[idx4[b]], rows[b], semg[b])

        def wait_gather(b):
            pltpu.make_async_copy(h2t_hbm.at[pl.ds(0, B)], rows[b],
                                  semg[b]).wait()

        def multiply(b):
            @pl.loop(0, B // 16)
            def _edge(g):
                evec = ev[b][pl.ds(g * 16, 16)]
                for i in range(16):
                    j = g * 16 + i
                    ef = jnp.full((16,), evec[i], jnp.float32)
                    eb = plsc.pack(ef, ef, format=plsc.PackFormat.INTERLEAVED)
                    rows[b][j, pl.ds(0, 32)] = rows[b][j, pl.ds(0, 32)] * eb

        load_prep(0, 0)
        load_prep(1, 1)

        @pl.loop(0, ROUNDS // 2)
        def _round(k):
            for b in range(2):
                rr = 2 * k + b
                wait_gather(b)
                multiply(b)
                pltpu.async_copy(rows[b], acc.at[idxd[b]], sems[b], add=True)

                @pl.when(rr + 2 < ROUNDS)
                def _():
                    pltpu.make_async_copy(rows[b], acc.at[idxd[b]],
                                          sems[b]).wait()
                    load_prep(rr + 2, b)

        pltpu.make_async_copy(rows[0], acc.at[idxd[0]], sems[0]).wait()
        pltpu.make_async_copy(rows[1], acc.at[idxd[1]], sems[1]).wait()
        plsc.subcore_barrier()
        pltpu.sync_copy(acc.at[pl.ds(s * RPT, RPT)],
                        msum_out.at[p, c, pl.ds(s * RPT, RPT)])
        plsc.subcore_barrier()


def _sc3(src, dst, e2t, h2t, zb):
    mesh = plsc.VectorSubcoreMesh(core_axis_name="c", subcore_axis_name="s")
    f = pl.kernel(
        _sc3_body,
        out_type=jax.ShapeDtypeStruct((4, NSC, NP, 32), jnp.bfloat16),
        mesh=mesh,
        compiler_params=pltpu.CompilerParams(use_tc_tiling_on_sc=False, needs_layout_passes=False),
        scratch_types=[
            pltpu.VMEM((B,), jnp.int32),
            pltpu.VMEM((B,), jnp.int32),
            pltpu.VMEM((B,), jnp.int32),
            pltpu.VMEM((B,), jnp.int32),
            pltpu.VMEM((B,), jnp.int32),
            pltpu.VMEM((B,), jnp.float32),
            pltpu.VMEM((B,), jnp.float32),
            pltpu.VMEM((B, 32), jnp.bfloat16),
            pltpu.VMEM((B, 32), jnp.bfloat16),
            pltpu.VMEM_SHARED((NP, 32), jnp.bfloat16),
            pltpu.SemaphoreType.DMA,
            pltpu.SemaphoreType.DMA,
            pltpu.SemaphoreType.DMA,
            pltpu.SemaphoreType.DMA,
        ],
    )
    return f(src, dst, e2t, h2t, zb)


def _tc_c_body(msum_ref, den_ref, b2_ref, out_ref):
    i = pl.program_id(0)
    m = msum_ref[...].astype(jnp.float32)
    den = den_ref[...]
    ms = m[:, 0] + m[:, 1]
    dn = den[0, :, 0:4] + den[1, :, 0:4] + 1e-16
    o = jnp.concatenate([ms[p] for p in range(4)], axis=1)
    kcol = lax.broadcasted_iota(jnp.int32, (4, 128), 1) // 32
    hrow = lax.broadcasted_iota(jnp.int32, (4, 128), 0)
    k4 = (kcol == hrow).astype(jnp.float32)
    dnrep = jnp.dot(dn, k4, preferred_element_type=jnp.float32,
                    precision=lax.Precision.HIGHEST)
    o = o / dnrep
    o = jnp.tanh(o + b2_ref[...][None])
    vid = i * 1024 + lax.broadcasted_iota(jnp.int32, (1024, 1), 0)
    o = jnp.where(vid < N, o, 0.0)
    psum = jnp.sum(o, axis=0, keepdims=True)

    @pl.when(i == 0)
    def _():
        out_ref[...] = psum

    @pl.when(i > 0)
    def _():
        out_ref[...] = out_ref[...] + psum

    @pl.when(i == NP // 1024 - 1)
    def _():
        out_ref[...] = out_ref[...] * (1.0 / N)


def _tc_c(msum, den2, b2):
    grid = (NP // 1024,)
    return pl.pallas_call(
        _tc_c_body,
        grid=grid,
        in_specs=[
            pl.BlockSpec((4, NSC, 1024, 32), lambda i: (0, 0, i, 0)),
            pl.BlockSpec((NSC, 1024, 8), lambda i: (0, i, 0)),
            pl.BlockSpec((128,), lambda i: (0,)),
        ],
        out_specs=pl.BlockSpec((1, 128), lambda i: (0, 0)),
        out_shape=jax.ShapeDtypeStruct((1, 128), jnp.float32),
    )(msum, den2, b2)


@jax.jit
def kernel(x, edge_index, W1, att_src1, att_dst1, b1, W2, att_src2,
           att_dst2, b2):
    ei = edge_index.astype(jnp.int32)
    loop = jnp.arange(N, dtype=jnp.int32)
    pad = jnp.full((EPP - EP,), TRASH, jnp.int32)
    src = jnp.concatenate([ei[0], loop, pad])
    dst = jnp.concatenate([ei[1], loop, pad])
    xflat = x[:, 0]
    xpad = jnp.pad(xflat, (0, NP - N), mode="edge")
    x2d = xpad.reshape(392, 128)
    z32 = jnp.zeros((NP, 32), jnp.float32)
    zb = jnp.zeros((NP, 32), jnp.bfloat16)

    c1 = _tc_a(x2d, W1.reshape(4, 16), att_src1[0], att_dst1[0])
    acc1 = _sc1(src, dst, xpad, c1, z32)
    h2p, a2c, c2 = _tc_b(acc1, W1[0], b1, W2,
                         att_src2.reshape(128), att_dst2.reshape(128))
    h2t = h2p.reshape(NP * 4, 32)
    e2t, den2 = _sc2(src, dst, a2c, c2, z32)
    msum = _sc3(src, dst, e2t, h2t, zb)
    return _tc_c(msum, den2, b2)


# speedup vs baseline: 140.4971x; 1.0010x over previous
import functools

import jax
import jax.numpy as jnp
from jax import lax
from jax.experimental import pallas as pl
from jax.experimental.pallas import tpu as pltpu
from jax.experimental.pallas import tpu_sc as plsc

N = 50000
E = 800000
EP = N + E
NP = 50176
N4P = NP * 4
TRASH = N
B = 1024
NSC = 2
NT = 16
ROUNDS = -(-EP // (NSC * NT * B))
EPP = ROUNDS * NSC * NT * B
PER_SC = EPP // NSC
RPT = NP // NT


def _tc_a_body(x2d_ref, w1r_ref, as1_ref, ad1_ref, c1_ref):
    x2d = x2d_ref[...]
    xmax = jnp.max(x2d)
    xmin = jnp.min(x2d)
    w1r = w1r_ref[...]
    s1 = jnp.sum(w1r * as1_ref[...], axis=1)
    d1 = jnp.sum(w1r * ad1_ref[...], axis=1)
    p1 = jnp.maximum(xmax * s1, xmin * s1)
    q1 = jnp.maximum(xmax * d1, xmin * d1)
    m = p1 + q1
    m1 = jnp.where(m > 0, m, 0.2 * m)
    rows = []
    for h in range(4):
        rows.append(jnp.full((1, 128), s1[h], jnp.float32))
    for h in range(4):
        rows.append(jnp.full((1, 128), d1[h], jnp.float32))
    for h in range(4):
        rows.append(jnp.full((1, 128), m1[h], jnp.float32))
    rows.append(jnp.zeros((4, 128), jnp.float32))
    c1_ref[...] = jnp.concatenate(rows, axis=0)


def _tc_a(x2d, w1r, as1r, ad1r):
    return pl.pallas_call(
        _tc_a_body,
        out_shape=jax.ShapeDtypeStruct((16, 128), jnp.float32),
    )(x2d, w1r, as1r, ad1r)


def _sc1_body(src_hbm, dst_hbm, xpad_hbm, c1_hbm, z_hbm, acc_out,
              idxs, idxd, xs, xd, rows, c1v, acc, sem):
    c = lax.axis_index("c")
    s = lax.axis_index("s")
    pltpu.sync_copy(z_hbm.at[pl.ds(s * RPT, RPT), pl.ds(0, 8)],
                    acc.at[pl.ds(s * RPT, RPT)])
    pltpu.sync_copy(c1_hbm, c1v)
    plsc.subcore_barrier()
    s1 = [c1v[h, pl.ds(0, 16)][0] for h in range(4)]
    d1 = [c1v[4 + h, pl.ds(0, 16)][0] for h in range(4)]
    m1 = [c1v[8 + h, pl.ds(0, 16)][0] for h in range(4)]
    iota = lax.broadcasted_iota(jnp.int32, (16,), 0)

    @pl.loop(0, ROUNDS)
    def _round(r):
        base = c * PER_SC + (r * NT + s) * B
        pltpu.sync_copy(src_hbm.at[pl.ds(base, B)], idxs)
        pltpu.sync_copy(dst_hbm.at[pl.ds(base, B)], idxd)
        pltpu.async_copy(xpad_hbm.at[idxs], xs, sem).wait()
        pltpu.async_copy(xpad_hbm.at[idxd], xd, sem).wait()

        @pl.loop(0, B // 16)
        def _grp(g):
            vs = xs[pl.ds(g * 16, 16)]
            vd = xd[pl.ds(g * 16, 16)]
            ridx = g * 16 + iota
            for h in range(4):
                a = vs * s1[h] + vd * d1[h]
                a = jnp.where(a > 0, a, 0.2 * a)
                e = jnp.exp(a - m1[h])
                hv = jnp.full((16,), h, jnp.int32)
                plsc.store_scatter(rows, [ridx, hv], e)
                plsc.store_scatter(rows, [ridx, hv + 4], e * vs)

        pltpu.sync_copy(rows, acc.at[idxd], add=True)

    plsc.subcore_barrier()
    pltpu.sync_copy(acc.at[pl.ds(s * RPT, RPT)],
                    acc_out.at[c, pl.ds(s * RPT, RPT)])


def _sc1(src, dst, xpad, c1, z32):
    mesh = plsc.VectorSubcoreMesh(core_axis_name="c", subcore_axis_name="s")
    f = pl.kernel(
        _sc1_body,
        out_type=jax.ShapeDtypeStruct((NSC, NP, 8), jnp.float32),
        mesh=mesh,
        compiler_params=pltpu.CompilerParams(use_tc_tiling_on_sc=False, needs_layout_passes=False),
        scratch_types=[
            pltpu.VMEM((B,), jnp.int32),
            pltpu.VMEM((B,), jnp.int32),
            pltpu.VMEM((B,), jnp.float32),
            pltpu.VMEM((B,), jnp.float32),
            pltpu.VMEM((B, 8), jnp.float32),
            pltpu.VMEM((16, 128), jnp.float32),
            pltpu.VMEM_SHARED((NP, 8), jnp.float32),
            pltpu.SemaphoreType.DMA,
        ],
    )
    return f(src, dst, xpad, c1, z32)


def _tc_b_body(acc1_ref, w1f_ref, b1_ref, w2_ref, as2f_ref, ad2f_ref,
               h2t_ref, a2c_ref, c2_ref):
    i = pl.program_id(0)
    a = acc1_ref[...]
    den = a[0, :, 0:4] + a[1, :, 0:4] + 1e-16
    ssum = a[0, :, 4:8] + a[1, :, 4:8]
    out1 = ssum / den
    kcol = lax.broadcasted_iota(jnp.int32, (4, 64), 1) // 16
    hrow = lax.broadcasted_iota(jnp.int32, (4, 64), 0)
    p4 = (kcol == hrow).astype(jnp.float32)
    h1 = jnp.dot(out1, p4, preferred_element_type=jnp.float32,
                 precision=lax.Precision.HIGHEST)
    h1 = h1 * w1f_ref[...][None] + b1_ref[...][None]
    h1 = jnp.tanh(h1)
    h2 = jnp.dot(h1, w2_ref[...], preferred_element_type=jnp.float32,
                 precision=lax.Precision.HIGHEST)
    for p in range(4):
        h2t_ref[p] = h2[:, p * 32:(p + 1) * 32].astype(jnp.bfloat16)
    kcol2 = lax.broadcasted_iota(jnp.int32, (128, 4), 0) // 32
    hrow2 = lax.broadcasted_iota(jnp.int32, (128, 4), 1)
    q4 = (kcol2 == hrow2).astype(jnp.float32)
    a2s = jnp.dot(h2 * as2f_ref[...][None], q4,
                  preferred_element_type=jnp.float32,
                  precision=lax.Precision.HIGHEST)
    a2d = jnp.dot(h2 * ad2f_ref[...][None], q4,
                  preferred_element_type=jnp.float32,
                  precision=lax.Precision.HIGHEST)
    a2c_ref[...] = jnp.concatenate([a2s, a2d], axis=1)
    pmax = jnp.max(a2s, axis=0)
    qmax = jnp.max(a2d, axis=0)
    rows = [jnp.full((1, 128), pmax[h], jnp.float32) for h in range(4)]
    rows += [jnp.full((1, 128), qmax[h], jnp.float32) for h in range(4)]
    cur = jnp.concatenate(rows, axis=0)

    @pl.when(i == 0)
    def _():
        c2_ref[...] = cur

    @pl.when(i > 0)
    def _():
        c2_ref[...] = jnp.maximum(c2_ref[...], cur)


def _tc_b(acc1, w1f, b1, w2, as2f, ad2f):
    grid = (NP // 1024,)
    return pl.pallas_call(
        _tc_b_body,
        grid=grid,
        in_specs=[
            pl.BlockSpec((NSC, 1024, 8), lambda i: (0, i, 0)),
            pl.BlockSpec((64,), lambda i: (0,)),
            pl.BlockSpec((64,), lambda i: (0,)),
            pl.BlockSpec((64, 128), lambda i: (0, 0)),
            pl.BlockSpec((128,), lambda i: (0,)),
            pl.BlockSpec((128,), lambda i: (0,)),
        ],
        out_specs=[
            pl.BlockSpec((4, 1024, 32), lambda i: (0, i, 0)),
            pl.BlockSpec((1024, 8), lambda i: (i, 0)),
            pl.BlockSpec((8, 128), lambda i: (0, 0)),
        ],
        out_shape=[
            jax.ShapeDtypeStruct((4, NP, 32), jnp.bfloat16),
            jax.ShapeDtypeStruct((NP, 8), jnp.float32),
            jax.ShapeDtypeStruct((8, 128), jnp.float32),
        ],
    )(acc1, w1f, b1, w2, as2f, ad2f)


def _sc2_body(src_hbm, dst_hbm, att_hbm, c2_hbm, z_hbm,
              e2t_out, den_out,
              idxs0, idxs1, idxd0, idxd1, asr0, asr1, adr0, adr1,
              rows0, rows1, e2b0, e2b1, c2v, acc,
              semg0, semg1, sems0, sems1, semw0, semw1):
    c = lax.axis_index("c")
    s = lax.axis_index("s")
    idxs = (idxs0, idxs1)
    idxd = (idxd0, idxd1)
    asr = (asr0, asr1)
    adr = (adr0, adr1)
    rows = (rows0, rows1)
    e2b = (e2b0, e2b1)
    semg = (semg0, semg1)
    sems = (sems0, sems1)
    semw = (semw0, semw1)
    pltpu.sync_copy(z_hbm.at[pl.ds(s * RPT, RPT), pl.ds(0, 8)],
                    acc.at[pl.ds(s * RPT, RPT)])
    pltpu.sync_copy(z_hbm.at[pl.ds(0, B), pl.ds(0, 8)], rows0)
    pltpu.sync_copy(z_hbm.at[pl.ds(0, B), pl.ds(0, 8)], rows1)
    pltpu.sync_copy(c2_hbm, c2v)
    plsc.subcore_barrier()
    m2 = []
    for h in range(4):
        mm = (c2v[h, pl.ds(0, 16)][0] + c2v[4 + h, pl.ds(0, 16)][0])
        m2.append(jnp.where(mm > 0, mm, 0.2 * mm))
    iota = lax.broadcasted_iota(jnp.int32, (16,), 0)

    def load_prep(rr, b):
        base = c * PER_SC + (rr * NT + s) * B
        pltpu.sync_copy(src_hbm.at[pl.ds(base, B)], idxs[b])
        pltpu.sync_copy(dst_hbm.at[pl.ds(base, B)], idxd[b])
        pltpu.async_copy(att_hbm.at[idxs[b]], asr[b], semg[b])
        pltpu.async_copy(att_hbm.at[idxd[b]], adr[b], semg[b])

    def compute(rr, b):
        base = c * PER_SC + (rr * NT + s) * B
        pltpu.make_async_copy(att_hbm.at[pl.ds(0, B)], asr[b], semg[b]).wait()
        pltpu.make_async_copy(att_hbm.at[pl.ds(0, B)], adr[b], semg[b]).wait()

        @pl.loop(0, B // 16)
        def _grp(g):
            ridx = g * 16 + iota
            for h in range(4):
                hv = jnp.full((16,), h, jnp.int32)
                av = plsc.load_gather(asr[b], [ridx, hv])
                bv = plsc.load_gather(adr[b], [ridx, hv + 4])
                a = av + bv
                a = jnp.where(a > 0, a, 0.2 * a)
                e = jnp.exp(a - m2[h])
                e2b[b][pl.ds(h * B + g * 16, 16)] = e
                plsc.store_scatter(rows[b], [ridx, hv], e)

        pltpu.async_copy(rows[b], acc.at[idxd[b]], sems[b], add=True)
        for h in range(4):
            pltpu.async_copy(e2b[b].at[pl.ds(h * B, B)],
                             e2t_out.at[h, pl.ds(base, B)], semw[b])

    def drain(rr, b):
        base = c * PER_SC + (rr * NT + s) * B
        pltpu.make_async_copy(rows[b], acc.at[idxd[b]], sems[b]).wait()
        for h in range(4):
            pltpu.make_async_copy(e2b[b].at[pl.ds(h * B, B)],
                                  e2t_out.at[h, pl.ds(base, B)],
                                  semw[b]).wait()

    load_prep(0, 0)
    load_prep(1, 1)

    @pl.loop(0, ROUNDS // 2)
    def _round(k):
        for b in range(2):
            rr = 2 * k + b
            compute(rr, b)

            @pl.when(rr + 2 < ROUNDS)
            def _():
                drain(rr, b)
                load_prep(rr + 2, b)

    drain(ROUNDS - 2, 0)
    drain(ROUNDS - 1, 1)
    plsc.subcore_barrier()
    pltpu.sync_copy(acc.at[pl.ds(s * RPT, RPT)],
                    den_out.at[c, pl.ds(s * RPT, RPT)])


def _sc2(src, dst, att, c2, z32):
    mesh = plsc.VectorSubcoreMesh(core_axis_name="c", subcore_axis_name="s")
    f = pl.kernel(
        _sc2_body,
        out_type=(
            jax.ShapeDtypeStruct((4, EPP), jnp.float32),
            jax.ShapeDtypeStruct((NSC, NP, 8), jnp.float32),
        ),
        mesh=mesh,
        compiler_params=pltpu.CompilerParams(use_tc_tiling_on_sc=False, needs_layout_passes=False),
        scratch_types=[
            pltpu.VMEM((B,), jnp.int32),
            pltpu.VMEM((B,), jnp.int32),
            pltpu.VMEM((B,), jnp.int32),
            pltpu.VMEM((B,), jnp.int32),
            pltpu.VMEM((B, 8), jnp.float32),
            pltpu.VMEM((B, 8), jnp.float32),
            pltpu.VMEM((B, 8), jnp.float32),
            pltpu.VMEM((B, 8), jnp.float32),
            pltpu.VMEM((B, 8), jnp.float32),
            pltpu.VMEM((B, 8), jnp.float32),
            pltpu.VMEM((4 * B,), jnp.float32),
            pltpu.VMEM((4 * B,), jnp.float32),
            pltpu.VMEM((8, 128), jnp.float32),
            pltpu.VMEM_SHARED((NP, 8), jnp.float32),
            pltpu.SemaphoreType.DMA,
            pltpu.SemaphoreType.DMA,
            pltpu.SemaphoreType.DMA,
            pltpu.SemaphoreType.DMA,
            pltpu.SemaphoreType.DMA,
            pltpu.SemaphoreType.DMA,
        ],
    )
    return f(src, dst, att, c2, z32)


def _sc3_body(src_hbm, dst_hbm, e2t_hbm, h2t_hbm, z_hbm, msum_out,
              idxs, idxd0, idxd1, idx40, idx41, ev0, ev1, rows0, rows1,
              acc, semg0, semg1, sems0, sems1):
    c = lax.axis_index("c")
    s = lax.axis_index("s")
    idxd = (idxd0, idxd1)
    idx4 = (idx40, idx41)
    ev = (ev0, ev1)
    rows = (rows0, rows1)
    semg = (semg0, semg1)
    sems = (sems0, sems1)

    for p in range(4):
        pltpu.sync_copy(z_hbm.at[pl.ds(s * RPT, RPT)],
                        acc.at[pl.ds(s * RPT, RPT)])
        plsc.subcore_barrier()

        def load_prep(rr, b):
            base = c * PER_SC + (rr * NT + s) * B
            pltpu.sync_copy(src_hbm.at[pl.ds(base, B)], idxs)
            pltpu.sync_copy(dst_hbm.at[pl.ds(base, B)], idxd[b])
            pltpu.sync_copy(e2t_hbm.at[p, pl.ds(base, B)], ev[b])

            @pl.loop(0, B // 16)
            def _gidx(g):
                iv = idxs[pl.ds(g * 16, 16)]
                idx4[b][pl.ds(g * 16, 16)] = iv + p * NP

            pltpu.async_copy(h2t_hbm.at[idx4[b]], rows[b], semg[b])

        def wait_gather(b):
            pltpu.make_async_copy(h2t_hbm.at[pl.ds(0, B)], rows[b],
                                  semg[b]).wait()

        def multiply(b):
            @pl.loop(0, B // 16)
            def _edge(g):
                evec = ev[b][pl.ds(g * 16, 16)]
                for i in range(16):
                    j = g * 16 + i
                    ef = jnp.full((16,), evec[i], jnp.float32)
                    eb = plsc.pack(ef, ef, format=plsc.PackFormat.INTERLEAVED)
                    rows[b][j, pl.ds(0, 32)] = rows[b][j, pl.ds(0, 32)] * eb

        load_prep(0, 0)
        load_prep(1, 1)

        @pl.loop(0, ROUNDS // 2)
        def _round(k):
            for b in range(2):
                rr = 2 * k + b
                wait_gather(b)
                multiply(b)
                pltpu.async_copy(rows[b], acc.at[idxd[b]], sems[b], add=True)

                @pl.when(rr + 2 < ROUNDS)
                def _():
                    pltpu.make_async_copy(rows[b], acc.at[idxd[b]],
                                          sems[b]).wait()
                    load_prep(rr + 2, b)

        pltpu.make_async_copy(rows[0], acc.at[idxd[0]], sems[0]).wait()
        pltpu.make_async_copy(rows[1], acc.at[idxd[1]], sems[1]).wait()
        plsc.subcore_barrier()
        pltpu.sync_copy(acc.at[pl.ds(s * RPT, RPT)],
                        msum_out.at[p, c, pl.ds(s * RPT, RPT)])
        plsc.subcore_barrier()


def _sc3(src, dst, e2t, h2t, zb):
    mesh = plsc.VectorSubcoreMesh(core_axis_name="c", subcore_axis_name="s")
    f = pl.kernel(
        _sc3_body,
        out_type=jax.ShapeDtypeStruct((4, NSC, NP, 32), jnp.bfloat16),
        mesh=mesh,
        compiler_params=pltpu.CompilerParams(use_tc_tiling_on_sc=False, needs_layout_passes=False),
        scratch_types=[
            pltpu.VMEM((B,), jnp.int32),
            pltpu.VMEM((B,), jnp.int32),
            pltpu.VMEM((B,), jnp.int32),
            pltpu.VMEM((B,), jnp.int32),
            pltpu.VMEM((B,), jnp.int32),
            pltpu.VMEM((B,), jnp.float32),
            pltpu.VMEM((B,), jnp.float32),
            pltpu.VMEM((B, 32), jnp.bfloat16),
            pltpu.VMEM((B, 32), jnp.bfloat16),
            pltpu.VMEM_SHARED((NP, 32), jnp.bfloat16),
            pltpu.SemaphoreType.DMA,
            pltpu.SemaphoreType.DMA,
            pltpu.SemaphoreType.DMA,
            pltpu.SemaphoreType.DMA,
        ],
    )
    return f(src, dst, e2t, h2t, zb)


def _tc_c_body(msum_ref, den_ref, b2_ref, out_ref):
    i = pl.program_id(0)
    m = msum_ref[...].astype(jnp.float32)
    den = den_ref[...]
    ms = m[:, 0] + m[:, 1]
    dn = den[0, :, 0:4] + den[1, :, 0:4] + 1e-16
    o = jnp.concatenate([ms[p] for p in range(4)], axis=1)
    kcol = lax.broadcasted_iota(jnp.int32, (4, 128), 1) // 32
    hrow = lax.broadcasted_iota(jnp.int32, (4, 128), 0)
    k4 = (kcol == hrow).astype(jnp.float32)
    dnrep = jnp.dot(dn, k4, preferred_element_type=jnp.float32,
                    precision=lax.Precision.HIGHEST)
    o = o / dnrep
    o = jnp.tanh(o + b2_ref[...][None])
    vid = i * 1024 + lax.broadcasted_iota(jnp.int32, (1024, 1), 0)
    o = jnp.where(vid < N, o, 0.0)
    psum = jnp.sum(o, axis=0, keepdims=True)

    @pl.when(i == 0)
    def _():
        out_ref[...] = psum

    @pl.when(i > 0)
    def _():
        out_ref[...] = out_ref[...] + psum

    @pl.when(i == NP // 1024 - 1)
    def _():
        out_ref[...] = out_ref[...] * (1.0 / N)


def _tc_c(msum, den2, b2):
    grid = (NP // 1024,)
    return pl.pallas_call(
        _tc_c_body,
        grid=grid,
        in_specs=[
            pl.BlockSpec((4, NSC, 1024, 32), lambda i: (0, 0, i, 0)),
            pl.BlockSpec((NSC, 1024, 8), lambda i: (0, i, 0)),
            pl.BlockSpec((128,), lambda i: (0,)),
        ],
        out_specs=pl.BlockSpec((1, 128), lambda i: (0, 0)),
        out_shape=jax.ShapeDtypeStruct((1, 128), jnp.float32),
    )(msum, den2, b2)


@jax.jit
def kernel(x, edge_index, W1, att_src1, att_dst1, b1, W2, att_src2,
           att_dst2, b2):
    ei = edge_index.astype(jnp.int32)
    loop = jnp.arange(N, dtype=jnp.int32)
    pad = jnp.full((EPP - EP,), TRASH, jnp.int32)
    src = jnp.concatenate([ei[0], loop, pad])
    dst = jnp.concatenate([ei[1], loop, pad])
    xflat = x[:, 0]
    xpad = jnp.pad(xflat, (0, NP - N), mode="edge")
    x2d = xpad.reshape(392, 128)
    z32 = jnp.zeros((NP, 32), jnp.float32)
    zb = jnp.zeros((NP, 32), jnp.bfloat16)

    c1 = _tc_a(x2d, W1.reshape(4, 16), att_src1[0], att_dst1[0])
    acc1 = _sc1(src, dst, xpad, c1, z32)
    h2p, a2c, c2 = _tc_b(acc1, W1[0], b1, W2,
                         att_src2.reshape(128), att_dst2.reshape(128))
    h2t = h2p.reshape(NP * 4, 32)
    e2t, den2 = _sc2(src, dst, a2c, c2, z32)
    msum = _sc3(src, dst, e2t, h2t, zb)
    return _tc_c(msum, den2, b2)
